# baseline (speedup 1.0000x reference)
import functools

import jax
import jax.numpy as jnp
from jax import lax
from jax.experimental import pallas as pl
from jax.experimental.pallas import tpu as pltpu
from jax.experimental.pallas import tpu_sc as plsc

N = 50000
E = 800000
FEA = 92
H = 64
EDIM = 41
L = 4
G = 256
CW = 16
TW = 2 * CW
NT = 4

K = 256
NCH = E // K
NSUB = 16
ROWS_PER_TILE = N // NSUB
ZCH = 625
MAXJ = (NCH + NSUB - 1) // NSUB

BE = 1000
BN = 400
NB_N = N // BN

_LOG1P_C = (
    0.9999751958009936, -0.4993836513699655, 0.32778470930088355,
    -0.22478407188994595, 0.13329712681080494, -0.0543128720320267,
    0.010571220016300703,
)


def _poly_log1p(u):
    r = jnp.float32(_LOG1P_C[-1])
    for c in _LOG1P_C[-2::-1]:
        r = r * u + jnp.float32(c)
    return r * u


_EXP_C = (1.0 / 720, 1.0 / 120, 1.0 / 24, 1.0 / 6, 0.5, 1.0, 1.0)


def _sc_exp(x):
    t = x * jnp.float32(1.4426950408889634)
    t = jnp.minimum(jnp.maximum(t, jnp.float32(-126.0)), jnp.float32(126.0))
    half = jnp.where(t >= 0, jnp.float32(0.5), jnp.float32(-0.5))
    n = (t + half).astype(jnp.int32)
    y = (t - n.astype(jnp.float32)) * jnp.float32(0.6931471805599453)
    p = jnp.float32(_EXP_C[0])
    for c in _EXP_C[1:]:
        p = p * y + jnp.float32(c)
    scale = lax.bitcast_convert_type((n + 127) << 23, jnp.float32)
    return p * scale


def _sc_rcp(d):
    r = lax.bitcast_convert_type(
        jnp.int32(0x7EF311C3) - lax.bitcast_convert_type(d, jnp.int32),
        jnp.float32)
    for _ in range(3):
        r = r * (jnp.float32(2.0) - d * r)
    return r


def _sc_softplus(x):
    u = _sc_exp(-jnp.abs(x))
    return jnp.maximum(x, jnp.float32(0.0)) + _poly_log1p(u)


def _sc_sigmoid(x):
    return _sc_rcp(jnp.float32(1.0) + _sc_exp(-x))


def _tc_softplus(x):
    return jnp.maximum(x, 0.0) + jnp.log1p(jnp.exp(-jnp.abs(x)))


def _cprep_body(ea_ref, w_ref, b_ref, out_ref):
    a = ea_ref[...]
    for l in range(L):
        for t in range(NT):
            out_ref[l, t] = (
                jnp.dot(a, w_ref[l, t], preferred_element_type=jnp.float32,
                        precision=lax.Precision.HIGHEST)
                + b_ref[l, t]
            )


def _cprep(ea, wc, bc):
    return pl.pallas_call(
        _cprep_body,
        grid=(E // BE,),
        in_specs=[
            pl.BlockSpec((BE, EDIM), lambda i: (i, 0)),
            pl.BlockSpec((L, NT, EDIM, TW), lambda i: (0, 0, 0, 0)),
            pl.BlockSpec((L, NT, 1, TW), lambda i: (0, 0, 0, 0)),
        ],
        out_specs=pl.BlockSpec((L, NT, BE, TW), lambda i: (0, 0, i, 0)),
        out_shape=jax.ShapeDtypeStruct((L, NT, E, TW), jnp.float32),
    )(ea, wc, bc)


def _embed_body(x_ref, we_ref, be_ref, wt1_ref, wt2_ref, h_ref, t1_ref, t2_ref):
    h = jnp.dot(x_ref[...], we_ref[...], preferred_element_type=jnp.float32,
                        precision=lax.Precision.HIGHEST)
    h = h + be_ref[...]
    h_ref[...] = h
    for t in range(NT):
        t1_ref[t] = jnp.dot(h, wt1_ref[t], preferred_element_type=jnp.float32,
                        precision=lax.Precision.HIGHEST)
        t2_ref[t] = jnp.dot(h, wt2_ref[t], preferred_element_type=jnp.float32,
                        precision=lax.Precision.HIGHEST)


def _embed(x, we, be, wt1, wt2):
    return pl.pallas_call(
        _embed_body,
        grid=(NB_N,),
        in_specs=[
            pl.BlockSpec((BN, FEA), lambda i: (i, 0)),
            pl.BlockSpec((FEA, H), lambda i: (0, 0)),
            pl.BlockSpec((1, H), lambda i: (0, 0)),
            pl.BlockSpec((NT, H, TW), lambda i: (0, 0, 0)),
            pl.BlockSpec((NT, H, TW), lambda i: (0, 0, 0)),
        ],
        out_specs=[
            pl.BlockSpec((BN, H), lambda i: (i, 0)),
            pl.BlockSpec((NT, BN, TW), lambda i: (0, i, 0)),
            pl.BlockSpec((NT, BN, TW), lambda i: (0, i, 0)),
        ],
        out_shape=[
            jax.ShapeDtypeStruct((N, H), jnp.float32),
            jax.ShapeDtypeStruct((NT, N, TW), jnp.float32),
            jax.ShapeDtypeStruct((NT, N, TW), jnp.float32),
        ],
    )(x, we, be, wt1, wt2)


def _stats_body(agg_ref, out_ref):
    i = pl.program_id(0)
    a = agg_ref[...]
    ssum = jnp.sum(a, axis=1)
    ssq = jnp.sum(a * a, axis=1)

    @pl.when(i == 0)
    def _():
        out_ref[0] = ssum
        out_ref[1] = ssq

    @pl.when(i > 0)
    def _():
        out_ref[0] += ssum
        out_ref[1] += ssq


def _stats(agg):
    return pl.pallas_call(
        _stats_body,
        grid=(NB_N,),
        in_specs=[pl.BlockSpec((NT, BN, CW), lambda i: (0, i, 0))],
        out_specs=pl.BlockSpec((2, NT, CW), lambda i: (0, 0, 0)),
        out_shape=jax.ShapeDtypeStruct((2, NT, CW), jnp.float32),
    )(agg)


def _bn_from_stats(stats_ref, agg_blk):
    ssum = stats_ref[0].reshape(1, H)
    ssq = stats_ref[1].reshape(1, H)
    mu = ssum * jnp.float32(1.0 / N)
    var = ssq * jnp.float32(1.0 / N) - mu * mu
    inv = lax.rsqrt(var + jnp.float32(1e-5))
    aggf = jnp.concatenate([agg_blk[t] for t in range(NT)], axis=1)
    return (aggf - mu) * inv


def _applyt_body(agg_ref, h_ref, st_ref, gam_ref, bet_ref, wt1_ref, wt2_ref,
                 hn_ref, t1_ref, t2_ref):
    norm = _bn_from_stats(st_ref, agg_ref[...])
    hn = h_ref[...] + gam_ref[...] * norm + bet_ref[...]
    hn_ref[...] = hn
    for t in range(NT):
        t1_ref[t] = jnp.dot(hn, wt1_ref[t], preferred_element_type=jnp.float32,
                        precision=lax.Precision.HIGHEST)
        t2_ref[t] = jnp.dot(hn, wt2_ref[t], preferred_element_type=jnp.float32,
                        precision=lax.Precision.HIGHEST)


def _apply_tables(agg, h, st, gam, bet, wt1, wt2):
    return pl.pallas_call(
        _applyt_body,
        grid=(NB_N,),
        in_specs=[
            pl.BlockSpec((NT, BN, CW), lambda i: (0, i, 0)),
            pl.BlockSpec((BN, H), lambda i: (i, 0)),
            pl.BlockSpec((2, NT, CW), lambda i: (0, 0, 0)),
            pl.BlockSpec((1, H), lambda i: (0, 0)),
            pl.BlockSpec((1, H), lambda i: (0, 0)),
            pl.BlockSpec((NT, H, TW), lambda i: (0, 0, 0)),
            pl.BlockSpec((NT, H, TW), lambda i: (0, 0, 0)),
        ],
        out_specs=[
            pl.BlockSpec((BN, H), lambda i: (i, 0)),
            pl.BlockSpec((NT, BN, TW), lambda i: (0, i, 0)),
            pl.BlockSpec((NT, BN, TW), lambda i: (0, i, 0)),
        ],
        out_shape=[
            jax.ShapeDtypeStruct((N, H), jnp.float32),
            jax.ShapeDtypeStruct((NT, N, TW), jnp.float32),
            jax.ShapeDtypeStruct((NT, N, TW), jnp.float32),
        ],
    )(agg, h, st, gam, bet, wt1, wt2)


def _applyl_body(agg_ref, h_ref, st_ref, gam_ref, bet_ref, hn_ref):
    norm = _bn_from_stats(st_ref, agg_ref[...])
    hn_ref[...] = h_ref[...] + gam_ref[...] * norm + bet_ref[...]


def _apply_last(agg, h, st, gam, bet):
    return pl.pallas_call(
        _applyl_body,
        grid=(NB_N,),
        in_specs=[
            pl.BlockSpec((NT, BN, CW), lambda i: (0, i, 0)),
            pl.BlockSpec((BN, H), lambda i: (i, 0)),
            pl.BlockSpec((2, NT, CW), lambda i: (0, 0, 0)),
            pl.BlockSpec((1, H), lambda i: (0, 0)),
            pl.BlockSpec((1, H), lambda i: (0, 0)),
        ],
        out_specs=pl.BlockSpec((BN, H), lambda i: (i, 0)),
        out_shape=jax.ShapeDtypeStruct((N, H), jnp.float32),
    )(agg, h, st, gam, bet)


def _pool_body(h_ref, b_ref, sums_ref, cnt_ref):
    i = pl.program_id(0)
    hblk = h_ref[...]
    ids = b_ref[0]
    gids = lax.broadcasted_iota(jnp.int32, (G, BN), 0)
    oh = (gids == ids).astype(jnp.float32)
    s = jnp.dot(oh, hblk, preferred_element_type=jnp.float32,
                        precision=lax.Precision.HIGHEST)
    c = jnp.sum(oh, axis=1, keepdims=True)

    @pl.when(i == 0)
    def _():
        sums_ref[...] = s
        cnt_ref[...] = c

    @pl.when(i > 0)
    def _():
        sums_ref[...] += s
        cnt_ref[...] += c


def _pool(h, batch3):
    return pl.pallas_call(
        _pool_body,
        grid=(NB_N,),
        in_specs=[
            pl.BlockSpec((BN, H), lambda i: (i, 0)),
            pl.BlockSpec((1, 1, BN), lambda i: (i, 0, 0)),
        ],
        out_specs=[
            pl.BlockSpec((G, H), lambda i: (0, 0)),
            pl.BlockSpec((G, 1), lambda i: (0, 0)),
        ],
        out_shape=[
            jax.ShapeDtypeStruct((G, H), jnp.float32),
            jax.ShapeDtypeStruct((G, 1), jnp.float32),
        ],
    )(h, batch3)


def _head_body(s_ref, c_ref, w1_ref, b1_ref, w2_ref, b2_ref, w3_ref, b3_ref,
               out_ref):
    pooled = s_ref[...] / jnp.maximum(c_ref[...], 1.0)
    y = _tc_softplus(
        jnp.dot(pooled, w1_ref[...], preferred_element_type=jnp.float32,
                        precision=lax.Precision.HIGHEST)
        + b1_ref[...])
    y = _tc_softplus(
        jnp.dot(y, w2_ref[...], preferred_element_type=jnp.float32,
                        precision=lax.Precision.HIGHEST)
        + b2_ref[...])
    out_ref[...] = (
        jnp.dot(y, w3_ref[...], preferred_element_type=jnp.float32,
                        precision=lax.Precision.HIGHEST)
        + b3_ref[...])


def _head(sums, cnt, w1, b1, w2, b2, w3, b3):
    return pl.pallas_call(
        _head_body,
        out_shape=jax.ShapeDtypeStruct((G, 1), jnp.float32),
    )(sums, cnt, w1, b1, w2, b2, w3, b3)


def _edge_body(t1_hbm, t2_hbm, c_hbm, dst_hbm, src_hbm, z_hbm, out_hbm,
               idxd, idxs, idxa1, idxa2, g1, g2, cb, mb, accum,
               sem1, sem2, sem3):
    c = lax.axis_index("c")
    s = lax.axis_index("s")
    rows0 = s * ROWS_PER_TILE

    for p in range(2):
        for k in range(ROWS_PER_TILE // ZCH):
            pltpu.sync_copy(z_hbm, accum.at[pl.ds(rows0 + k * ZCH, ZCH)])
        plsc.subcore_barrier()

        tN = (p * 2) * N + c * N

        def chunk_body(j, carry):
            ch = s + NSUB * j

            @pl.when(ch < NCH)
            def _():
                off = ch * K
                pltpu.sync_copy(dst_hbm.at[pl.ds(off, K)], idxd)
                pltpu.sync_copy(src_hbm.at[pl.ds(off, K)], idxs)

                def adj(i, carry2):
                    idxa1[pl.ds(i * 16, 16)] = idxd[pl.ds(i * 16, 16)] + tN
                    idxa2[pl.ds(i * 16, 16)] = idxs[pl.ds(i * 16, 16)] + tN
                    return carry2

                lax.fori_loop(0, K // 16, adj, 0)

                cp1 = pltpu.async_copy(t1_hbm.at[idxa1], g1, sem1)
                cp2 = pltpu.async_copy(t2_hbm.at[idxa2], g2, sem2)
                cp3 = pltpu.async_copy(
                    c_hbm.at[p * 2 + c, pl.ds(off, K)], cb, sem3)
                cp1.wait()
                cp2.wait()
                cp3.wait()

                def edge(e, carry2):
                    pf = (g1[e, pl.ds(0, CW)] + g2[e, pl.ds(0, CW)]
                          + cb[e, pl.ds(0, CW)])
                    ps = (g1[e, pl.ds(CW, CW)] + g2[e, pl.ds(CW, CW)]
                          + cb[e, pl.ds(CW, CW)])
                    mb[e, pl.ds(0, CW)] = _sc_sigmoid(pf) * _sc_softplus(ps)
                    return carry2

                lax.fori_loop(0, K, edge, 0)
                pltpu.sync_copy(mb, accum.at[idxd], add=True)

            return carry

        lax.fori_loop(0, MAXJ, chunk_body, 0)
        plsc.subcore_barrier()
        pltpu.sync_copy(accum.at[pl.ds(rows0, ROWS_PER_TILE)],
                        out_hbm.at[p, c, s])


@functools.lru_cache(maxsize=None)
def _make_edge_kernel():
    return pl.kernel(
        _edge_body,
        mesh=plsc.VectorSubcoreMesh(core_axis_name="c", subcore_axis_name="s"),
        compiler_params=pltpu.CompilerParams(use_tc_tiling_on_sc=False),
        out_type=jax.ShapeDtypeStruct((2, 2, NSUB, ROWS_PER_TILE, CW),
                                      jnp.float32),
        scratch_types=[
            pltpu.VMEM((K,), jnp.int32),
            pltpu.VMEM((K,), jnp.int32),
            pltpu.VMEM((K,), jnp.int32),
            pltpu.VMEM((K,), jnp.int32),
            pltpu.VMEM((K, TW), jnp.float32),
            pltpu.VMEM((K, TW), jnp.float32),
            pltpu.VMEM((K, TW), jnp.float32),
            pltpu.VMEM((K, CW), jnp.float32),
            pltpu.VMEM_SHARED((N, CW), jnp.float32),
            pltpu.SemaphoreType.DMA,
            pltpu.SemaphoreType.DMA,
            pltpu.SemaphoreType.DMA,
        ],
    )


def _edge_stage(t1, t2, cl, dst, src, zeros):
    out = _make_edge_kernel()(t1, t2, cl, dst, src, zeros)
    return out.reshape(NT, N, CW)


def kernel(x, edge_index, edge_attr, batch, W_emb, b_emb, Wf, bf, Ws, bs,
           gamma, beta, W1, b1, W2, b2, W3, b3):
    f32 = jnp.float32
    src = edge_index[0].astype(jnp.int32)
    dst = edge_index[1].astype(jnp.int32)

    wt1, wt2, wc, bc = [], [], [], []
    for l in range(L):
        w_f, w_s = Wf[l], Ws[l]
        q = lambda w, r0, t: w[r0:r0 + H, t * CW:(t + 1) * CW]
        wt1.append(jnp.stack([
            jnp.concatenate([q(w_f, 0, t), q(w_s, 0, t)], axis=1)
            for t in range(NT)]))
        wt2.append(jnp.stack([
            jnp.concatenate([q(w_f, H, t), q(w_s, H, t)], axis=1)
            for t in range(NT)]))
        wc.append(jnp.stack([
            jnp.concatenate([w_f[2 * H:, t * CW:(t + 1) * CW],
                             w_s[2 * H:, t * CW:(t + 1) * CW]], axis=1)
            for t in range(NT)]))
        bc.append(jnp.stack([
            jnp.concatenate([bf[l][t * CW:(t + 1) * CW],
                             bs[l][t * CW:(t + 1) * CW]])
            for t in range(NT)])[:, None, :])
    wt1 = jnp.stack(wt1)
    wt2 = jnp.stack(wt2)
    wc = jnp.stack(wc)
    bc = jnp.stack(bc)

    c_all = _cprep(edge_attr.astype(f32), wc, bc)
    zeros = jnp.zeros((ZCH, CW), f32)
    batch3 = batch.astype(jnp.int32).reshape(NB_N, 1, BN)

    h, t1, t2 = _embed(x.astype(f32), W_emb, b_emb.reshape(1, H),
                       wt1[0], wt2[0])
    for l in range(L):
        agg = _edge_stage(t1.reshape(NT * N, TW), t2.reshape(NT * N, TW),
                          c_all[l], dst, src, zeros)
        st = _stats(agg)
        gam = gamma[l].reshape(1, H)
        bet = beta[l].reshape(1, H)
        if l < L - 1:
            h, t1, t2 = _apply_tables(agg, h, st, gam, bet,
                                      wt1[l + 1], wt2[l + 1])
        else:
            h = _apply_last(agg, h, st, gam, bet)

    sums, cnt = _pool(h, batch3)
    return _head(sums, cnt, W1, b1.reshape(1, 128), W2, b2.reshape(1, 64),
                 W3, b3.reshape(1, 1))


# speedup vs baseline: 1.4738x; 1.4738x over previous
import functools

import jax
import jax.numpy as jnp
from jax import lax
from jax.experimental import pallas as pl
from jax.experimental.pallas import tpu as pltpu
from jax.experimental.pallas import tpu_sc as plsc

N = 50000
E = 800000
FEA = 92
H = 64
EDIM = 41
L = 4
G = 256
CW = 16
TW = 2 * CW
NT = 4

K = 256
NCH = E // K
NSUB = 16
ROWS_PER_TILE = N // NSUB
ZCH = 625
MAXJ = (NCH + NSUB - 1) // NSUB

BE = 1000
BN = 400
NB_N = N // BN

_LOG1P_C = (
    0.9999751958009936, -0.4993836513699655, 0.32778470930088355,
    -0.22478407188994595, 0.13329712681080494, -0.0543128720320267,
    0.010571220016300703,
)


def _poly_log1p(u):
    r = jnp.float32(_LOG1P_C[-1])
    for c in _LOG1P_C[-2::-1]:
        r = r * u + jnp.float32(c)
    return r * u


_EXP_C = (1.0 / 720, 1.0 / 120, 1.0 / 24, 1.0 / 6, 0.5, 1.0, 1.0)


def _sc_exp(x):
    t = x * jnp.float32(1.4426950408889634)
    t = jnp.minimum(jnp.maximum(t, jnp.float32(-126.0)), jnp.float32(126.0))
    half = jnp.where(t >= 0, jnp.float32(0.5), jnp.float32(-0.5))
    n = (t + half).astype(jnp.int32)
    y = (t - n.astype(jnp.float32)) * jnp.float32(0.6931471805599453)
    p = jnp.float32(_EXP_C[0])
    for c in _EXP_C[1:]:
        p = p * y + jnp.float32(c)
    scale = lax.bitcast_convert_type((n + 127) << 23, jnp.float32)
    return p * scale


def _sc_rcp(d):
    r = lax.bitcast_convert_type(
        jnp.int32(0x7EF311C3) - lax.bitcast_convert_type(d, jnp.int32),
        jnp.float32)
    for _ in range(3):
        r = r * (jnp.float32(2.0) - d * r)
    return r


def _sc_softplus(x):
    u = _sc_exp(-jnp.abs(x))
    return jnp.maximum(x, jnp.float32(0.0)) + _poly_log1p(u)


def _sc_sigmoid(x):
    return _sc_rcp(jnp.float32(1.0) + _sc_exp(-x))


def _tc_softplus(x):
    return jnp.maximum(x, 0.0) + jnp.log1p(jnp.exp(-jnp.abs(x)))


def _cprep_body(ea_ref, w_ref, b_ref, out_ref):
    a = ea_ref[...]
    for l in range(L):
        for t in range(NT):
            out_ref[l, t] = (
                jnp.dot(a, w_ref[l, t], preferred_element_type=jnp.float32,
                        precision=lax.Precision.HIGHEST)
                + b_ref[l, t]
            )


def _cprep(ea, wc, bc):
    return pl.pallas_call(
        _cprep_body,
        grid=(E // BE,),
        in_specs=[
            pl.BlockSpec((BE, EDIM), lambda i: (i, 0)),
            pl.BlockSpec((L, NT, EDIM, TW), lambda i: (0, 0, 0, 0)),
            pl.BlockSpec((L, NT, 1, TW), lambda i: (0, 0, 0, 0)),
        ],
        out_specs=pl.BlockSpec((L, NT, BE, TW), lambda i: (0, 0, i, 0)),
        out_shape=jax.ShapeDtypeStruct((L, NT, E, TW), jnp.float32),
    )(ea, wc, bc)


def _embed_body(x_ref, we_ref, be_ref, wt1_ref, wt2_ref, h_ref, t1_ref, t2_ref):
    h = jnp.dot(x_ref[...], we_ref[...], preferred_element_type=jnp.float32,
                        precision=lax.Precision.HIGHEST)
    h = h + be_ref[...]
    h_ref[...] = h
    for t in range(NT):
        t1_ref[t] = jnp.dot(h, wt1_ref[t], preferred_element_type=jnp.float32,
                        precision=lax.Precision.HIGHEST)
        t2_ref[t] = jnp.dot(h, wt2_ref[t], preferred_element_type=jnp.float32,
                        precision=lax.Precision.HIGHEST)


def _embed(x, we, be, wt1, wt2):
    return pl.pallas_call(
        _embed_body,
        grid=(NB_N,),
        in_specs=[
            pl.BlockSpec((BN, FEA), lambda i: (i, 0)),
            pl.BlockSpec((FEA, H), lambda i: (0, 0)),
            pl.BlockSpec((1, H), lambda i: (0, 0)),
            pl.BlockSpec((NT, H, TW), lambda i: (0, 0, 0)),
            pl.BlockSpec((NT, H, TW), lambda i: (0, 0, 0)),
        ],
        out_specs=[
            pl.BlockSpec((BN, H), lambda i: (i, 0)),
            pl.BlockSpec((NT, BN, TW), lambda i: (0, i, 0)),
            pl.BlockSpec((NT, BN, TW), lambda i: (0, i, 0)),
        ],
        out_shape=[
            jax.ShapeDtypeStruct((N, H), jnp.float32),
            jax.ShapeDtypeStruct((NT, N, TW), jnp.float32),
            jax.ShapeDtypeStruct((NT, N, TW), jnp.float32),
        ],
    )(x, we, be, wt1, wt2)


def _stats_body(agg_ref, out_ref):
    i = pl.program_id(0)
    a = agg_ref[...]
    ssum = jnp.sum(a, axis=1)
    ssq = jnp.sum(a * a, axis=1)

    @pl.when(i == 0)
    def _():
        out_ref[0] = ssum
        out_ref[1] = ssq

    @pl.when(i > 0)
    def _():
        out_ref[0] += ssum
        out_ref[1] += ssq


def _stats(agg):
    return pl.pallas_call(
        _stats_body,
        grid=(NB_N,),
        in_specs=[pl.BlockSpec((NT, BN, CW), lambda i: (0, i, 0))],
        out_specs=pl.BlockSpec((2, NT, CW), lambda i: (0, 0, 0)),
        out_shape=jax.ShapeDtypeStruct((2, NT, CW), jnp.float32),
    )(agg)


def _bn_from_stats(stats_ref, agg_blk):
    ssum = stats_ref[0].reshape(1, H)
    ssq = stats_ref[1].reshape(1, H)
    mu = ssum * jnp.float32(1.0 / N)
    var = ssq * jnp.float32(1.0 / N) - mu * mu
    inv = lax.rsqrt(var + jnp.float32(1e-5))
    aggf = jnp.concatenate([agg_blk[t] for t in range(NT)], axis=1)
    return (aggf - mu) * inv


def _applyt_body(agg_ref, h_ref, st_ref, gam_ref, bet_ref, wt1_ref, wt2_ref,
                 hn_ref, t1_ref, t2_ref):
    norm = _bn_from_stats(st_ref, agg_ref[...])
    hn = h_ref[...] + gam_ref[...] * norm + bet_ref[...]
    hn_ref[...] = hn
    for t in range(NT):
        t1_ref[t] = jnp.dot(hn, wt1_ref[t], preferred_element_type=jnp.float32,
                        precision=lax.Precision.HIGHEST)
        t2_ref[t] = jnp.dot(hn, wt2_ref[t], preferred_element_type=jnp.float32,
                        precision=lax.Precision.HIGHEST)


def _apply_tables(agg, h, st, gam, bet, wt1, wt2):
    return pl.pallas_call(
        _applyt_body,
        grid=(NB_N,),
        in_specs=[
            pl.BlockSpec((NT, BN, CW), lambda i: (0, i, 0)),
            pl.BlockSpec((BN, H), lambda i: (i, 0)),
            pl.BlockSpec((2, NT, CW), lambda i: (0, 0, 0)),
            pl.BlockSpec((1, H), lambda i: (0, 0)),
            pl.BlockSpec((1, H), lambda i: (0, 0)),
            pl.BlockSpec((NT, H, TW), lambda i: (0, 0, 0)),
            pl.BlockSpec((NT, H, TW), lambda i: (0, 0, 0)),
        ],
        out_specs=[
            pl.BlockSpec((BN, H), lambda i: (i, 0)),
            pl.BlockSpec((NT, BN, TW), lambda i: (0, i, 0)),
            pl.BlockSpec((NT, BN, TW), lambda i: (0, i, 0)),
        ],
        out_shape=[
            jax.ShapeDtypeStruct((N, H), jnp.float32),
            jax.ShapeDtypeStruct((NT, N, TW), jnp.float32),
            jax.ShapeDtypeStruct((NT, N, TW), jnp.float32),
        ],
    )(agg, h, st, gam, bet, wt1, wt2)


def _applyl_body(agg_ref, h_ref, st_ref, gam_ref, bet_ref, hn_ref):
    norm = _bn_from_stats(st_ref, agg_ref[...])
    hn_ref[...] = h_ref[...] + gam_ref[...] * norm + bet_ref[...]


def _apply_last(agg, h, st, gam, bet):
    return pl.pallas_call(
        _applyl_body,
        grid=(NB_N,),
        in_specs=[
            pl.BlockSpec((NT, BN, CW), lambda i: (0, i, 0)),
            pl.BlockSpec((BN, H), lambda i: (i, 0)),
            pl.BlockSpec((2, NT, CW), lambda i: (0, 0, 0)),
            pl.BlockSpec((1, H), lambda i: (0, 0)),
            pl.BlockSpec((1, H), lambda i: (0, 0)),
        ],
        out_specs=pl.BlockSpec((BN, H), lambda i: (i, 0)),
        out_shape=jax.ShapeDtypeStruct((N, H), jnp.float32),
    )(agg, h, st, gam, bet)


def _pool_body(h_ref, b_ref, sums_ref, cnt_ref):
    i = pl.program_id(0)
    hblk = h_ref[...]
    ids = b_ref[0]
    gids = lax.broadcasted_iota(jnp.int32, (G, BN), 0)
    oh = (gids == ids).astype(jnp.float32)
    s = jnp.dot(oh, hblk, preferred_element_type=jnp.float32,
                        precision=lax.Precision.HIGHEST)
    c = jnp.sum(oh, axis=1, keepdims=True)

    @pl.when(i == 0)
    def _():
        sums_ref[...] = s
        cnt_ref[...] = c

    @pl.when(i > 0)
    def _():
        sums_ref[...] += s
        cnt_ref[...] += c


def _pool(h, batch3):
    return pl.pallas_call(
        _pool_body,
        grid=(NB_N,),
        in_specs=[
            pl.BlockSpec((BN, H), lambda i: (i, 0)),
            pl.BlockSpec((1, 1, BN), lambda i: (i, 0, 0)),
        ],
        out_specs=[
            pl.BlockSpec((G, H), lambda i: (0, 0)),
            pl.BlockSpec((G, 1), lambda i: (0, 0)),
        ],
        out_shape=[
            jax.ShapeDtypeStruct((G, H), jnp.float32),
            jax.ShapeDtypeStruct((G, 1), jnp.float32),
        ],
    )(h, batch3)


def _head_body(s_ref, c_ref, w1_ref, b1_ref, w2_ref, b2_ref, w3_ref, b3_ref,
               out_ref):
    pooled = s_ref[...] / jnp.maximum(c_ref[...], 1.0)
    y = _tc_softplus(
        jnp.dot(pooled, w1_ref[...], preferred_element_type=jnp.float32,
                        precision=lax.Precision.HIGHEST)
        + b1_ref[...])
    y = _tc_softplus(
        jnp.dot(y, w2_ref[...], preferred_element_type=jnp.float32,
                        precision=lax.Precision.HIGHEST)
        + b2_ref[...])
    out_ref[...] = (
        jnp.dot(y, w3_ref[...], preferred_element_type=jnp.float32,
                        precision=lax.Precision.HIGHEST)
        + b3_ref[...])


def _head(sums, cnt, w1, b1, w2, b2, w3, b3):
    return pl.pallas_call(
        _head_body,
        out_shape=jax.ShapeDtypeStruct((G, 1), jnp.float32),
    )(sums, cnt, w1, b1, w2, b2, w3, b3)


def _edge_body(t1_hbm, t2_hbm, c_hbm, dst_hbm, src_hbm, z_hbm, out_hbm,
               idxd, idxs, idxa1, idxa2, g1, g2, cb, mb, accum,
               sem1, sem2, sem3):
    c = lax.axis_index("c")
    s = lax.axis_index("s")
    rows0 = s * ROWS_PER_TILE

    for p in range(2):
        for k in range(ROWS_PER_TILE // ZCH):
            pltpu.sync_copy(z_hbm, accum.at[pl.ds(rows0 + k * ZCH, ZCH)])
        plsc.subcore_barrier()

        tN = (p * 2) * N + c * N

        def chunk_body(j, carry):
            ch = s + NSUB * j

            @pl.when(ch < NCH)
            def _():
                off = ch * K
                pltpu.sync_copy(dst_hbm.at[pl.ds(off, K)], idxd)
                pltpu.sync_copy(src_hbm.at[pl.ds(off, K)], idxs)

                @plsc.parallel_loop(0, K, 16, unroll=4)
                def _adj(i):
                    idxa1[pl.ds(i, 16)] = idxd[pl.ds(i, 16)] + tN
                    idxa2[pl.ds(i, 16)] = idxs[pl.ds(i, 16)] + tN

                cp1 = pltpu.async_copy(t1_hbm.at[idxa1], g1, sem1)
                cp2 = pltpu.async_copy(t2_hbm.at[idxa2], g2, sem2)
                cp3 = pltpu.async_copy(
                    c_hbm.at[p * 2 + c, pl.ds(off, K)], cb, sem3)
                cp1.wait()
                cp2.wait()
                cp3.wait()

                @plsc.parallel_loop(0, K, 1, unroll=8)
                def _edge(e):
                    pf = (g1[e, pl.ds(0, CW)] + g2[e, pl.ds(0, CW)]
                          + cb[e, pl.ds(0, CW)])
                    ps = (g1[e, pl.ds(CW, CW)] + g2[e, pl.ds(CW, CW)]
                          + cb[e, pl.ds(CW, CW)])
                    mb[e, pl.ds(0, CW)] = _sc_sigmoid(pf) * _sc_softplus(ps)
                pltpu.sync_copy(mb, accum.at[idxd], add=True)

            return carry

        lax.fori_loop(0, MAXJ, chunk_body, 0)
        plsc.subcore_barrier()
        pltpu.sync_copy(accum.at[pl.ds(rows0, ROWS_PER_TILE)],
                        out_hbm.at[p, c, s])


@functools.lru_cache(maxsize=None)
def _make_edge_kernel():
    return pl.kernel(
        _edge_body,
        mesh=plsc.VectorSubcoreMesh(core_axis_name="c", subcore_axis_name="s"),
        compiler_params=pltpu.CompilerParams(use_tc_tiling_on_sc=False),
        out_type=jax.ShapeDtypeStruct((2, 2, NSUB, ROWS_PER_TILE, CW),
                                      jnp.float32),
        scratch_types=[
            pltpu.VMEM((K,), jnp.int32),
            pltpu.VMEM((K,), jnp.int32),
            pltpu.VMEM((K,), jnp.int32),
            pltpu.VMEM((K,), jnp.int32),
            pltpu.VMEM((K, TW), jnp.float32),
            pltpu.VMEM((K, TW), jnp.float32),
            pltpu.VMEM((K, TW), jnp.float32),
            pltpu.VMEM((K, CW), jnp.float32),
            pltpu.VMEM_SHARED((N, CW), jnp.float32),
            pltpu.SemaphoreType.DMA,
            pltpu.SemaphoreType.DMA,
            pltpu.SemaphoreType.DMA,
        ],
    )


def _edge_stage(t1, t2, cl, dst, src, zeros):
    out = _make_edge_kernel()(t1, t2, cl, dst, src, zeros)
    return out.reshape(NT, N, CW)


def kernel(x, edge_index, edge_attr, batch, W_emb, b_emb, Wf, bf, Ws, bs,
           gamma, beta, W1, b1, W2, b2, W3, b3):
    f32 = jnp.float32
    src = edge_index[0].astype(jnp.int32)
    dst = edge_index[1].astype(jnp.int32)

    wt1, wt2, wc, bc = [], [], [], []
    for l in range(L):
        w_f, w_s = Wf[l], Ws[l]
        q = lambda w, r0, t: w[r0:r0 + H, t * CW:(t + 1) * CW]
        wt1.append(jnp.stack([
            jnp.concatenate([q(w_f, 0, t), q(w_s, 0, t)], axis=1)
            for t in range(NT)]))
        wt2.append(jnp.stack([
            jnp.concatenate([q(w_f, H, t), q(w_s, H, t)], axis=1)
            for t in range(NT)]))
        wc.append(jnp.stack([
            jnp.concatenate([w_f[2 * H:, t * CW:(t + 1) * CW],
                             w_s[2 * H:, t * CW:(t + 1) * CW]], axis=1)
            for t in range(NT)]))
        bc.append(jnp.stack([
            jnp.concatenate([bf[l][t * CW:(t + 1) * CW],
                             bs[l][t * CW:(t + 1) * CW]])
            for t in range(NT)])[:, None, :])
    wt1 = jnp.stack(wt1)
    wt2 = jnp.stack(wt2)
    wc = jnp.stack(wc)
    bc = jnp.stack(bc)

    c_all = _cprep(edge_attr.astype(f32), wc, bc)
    zeros = jnp.zeros((ZCH, CW), f32)
    batch3 = batch.astype(jnp.int32).reshape(NB_N, 1, BN)

    h, t1, t2 = _embed(x.astype(f32), W_emb, b_emb.reshape(1, H),
                       wt1[0], wt2[0])
    for l in range(L):
        agg = _edge_stage(t1.reshape(NT * N, TW), t2.reshape(NT * N, TW),
                          c_all[l], dst, src, zeros)
        st = _stats(agg)
        gam = gamma[l].reshape(1, H)
        bet = beta[l].reshape(1, H)
        if l < L - 1:
            h, t1, t2 = _apply_tables(agg, h, st, gam, bet,
                                      wt1[l + 1], wt2[l + 1])
        else:
            h = _apply_last(agg, h, st, gam, bet)

    sums, cnt = _pool(h, batch3)
    return _head(sums, cnt, W1, b1.reshape(1, 128), W2, b2.reshape(1, 64),
                 W3, b3.reshape(1, 1))


# speedup vs baseline: 2.7031x; 1.8342x over previous
import functools

import jax
import jax.numpy as jnp
from jax import lax
from jax.experimental import pallas as pl
from jax.experimental.pallas import tpu as pltpu
from jax.experimental.pallas import tpu_sc as plsc

N = 50000
E = 800000
FEA = 92
H = 64
EDIM = 41
L = 4
G = 256
CW = 16
TW = 2 * CW
NT = 4

K = 256
NCH = E // K
NSUB = 16
ROWS_PER_TILE = N // NSUB
ZCH = 625
MAXJ = (NCH + NSUB - 1) // NSUB

BE = 1000
BN = 400
NB_N = N // BN

_LOG1P_C = (
    0.9999751958009936, -0.4993836513699655, 0.32778470930088355,
    -0.22478407188994595, 0.13329712681080494, -0.0543128720320267,
    0.010571220016300703,
)


def _poly_log1p(u):
    r = jnp.float32(_LOG1P_C[-1])
    for c in _LOG1P_C[-2::-1]:
        r = r * u + jnp.float32(c)
    return r * u


_EXP_C = (1.0 / 720, 1.0 / 120, 1.0 / 24, 1.0 / 6, 0.5, 1.0, 1.0)


def _sc_exp(x):
    t = x * jnp.float32(1.4426950408889634)
    t = jnp.minimum(jnp.maximum(t, jnp.float32(-126.0)), jnp.float32(126.0))
    half = jnp.where(t >= 0, jnp.float32(0.5), jnp.float32(-0.5))
    n = (t + half).astype(jnp.int32)
    y = (t - n.astype(jnp.float32)) * jnp.float32(0.6931471805599453)
    p = jnp.float32(_EXP_C[0])
    for c in _EXP_C[1:]:
        p = p * y + jnp.float32(c)
    scale = lax.bitcast_convert_type((n + 127) << 23, jnp.float32)
    return p * scale


def _sc_rcp(d):
    r = lax.bitcast_convert_type(
        jnp.int32(0x7EF311C3) - lax.bitcast_convert_type(d, jnp.int32),
        jnp.float32)
    for _ in range(3):
        r = r * (jnp.float32(2.0) - d * r)
    return r


def _sc_softplus(x):
    u = _sc_exp(-jnp.abs(x))
    return jnp.maximum(x, jnp.float32(0.0)) + _poly_log1p(u)


def _sc_sigmoid(x):
    return _sc_rcp(jnp.float32(1.0) + _sc_exp(-x))


def _tc_softplus(x):
    return jnp.maximum(x, 0.0) + jnp.log1p(jnp.exp(-jnp.abs(x)))


_CJ = 2 * L * TW


def _cprep_body(ea_ref, w_ref, b_ref, out_ref):
    r = jnp.dot(ea_ref[...], w_ref[...], preferred_element_type=jnp.float32,
                precision=lax.Precision.HIGHEST) + b_ref[...]
    out_ref[0] = r[:, :_CJ]
    out_ref[1] = r[:, _CJ:]


def _cprep(ea, wcat, bcat):
    return pl.pallas_call(
        _cprep_body,
        grid=(E // BE,),
        in_specs=[
            pl.BlockSpec((BE, EDIM), lambda i: (i, 0)),
            pl.BlockSpec((EDIM, 2 * _CJ), lambda i: (0, 0)),
            pl.BlockSpec((1, 2 * _CJ), lambda i: (0, 0)),
        ],
        out_specs=pl.BlockSpec((2, BE, _CJ), lambda i: (0, i, 0)),
        out_shape=jax.ShapeDtypeStruct((2, E, _CJ), jnp.float32),
    )(ea, wcat, bcat)


def _embed_body(x_ref, we_ref, be_ref, wt1_ref, wt2_ref, h_ref, t1_ref, t2_ref):
    h = jnp.dot(x_ref[...], we_ref[...], preferred_element_type=jnp.float32,
                        precision=lax.Precision.HIGHEST)
    h = h + be_ref[...]
    h_ref[...] = h
    for t in range(NT):
        t1_ref[t] = jnp.dot(h, wt1_ref[t], preferred_element_type=jnp.float32,
                        precision=lax.Precision.HIGHEST)
        t2_ref[t] = jnp.dot(h, wt2_ref[t], preferred_element_type=jnp.float32,
                        precision=lax.Precision.HIGHEST)


def _embed(x, we, be, wt1, wt2):
    return pl.pallas_call(
        _embed_body,
        grid=(NB_N,),
        in_specs=[
            pl.BlockSpec((BN, FEA), lambda i: (i, 0)),
            pl.BlockSpec((FEA, H), lambda i: (0, 0)),
            pl.BlockSpec((1, H), lambda i: (0, 0)),
            pl.BlockSpec((NT, H, TW), lambda i: (0, 0, 0)),
            pl.BlockSpec((NT, H, TW), lambda i: (0, 0, 0)),
        ],
        out_specs=[
            pl.BlockSpec((BN, H), lambda i: (i, 0)),
            pl.BlockSpec((NT, BN, TW), lambda i: (0, i, 0)),
            pl.BlockSpec((NT, BN, TW), lambda i: (0, i, 0)),
        ],
        out_shape=[
            jax.ShapeDtypeStruct((N, H), jnp.float32),
            jax.ShapeDtypeStruct((NT, N, TW), jnp.float32),
            jax.ShapeDtypeStruct((NT, N, TW), jnp.float32),
        ],
    )(x, we, be, wt1, wt2)


def _stats_body(agg_ref, out_ref):
    i = pl.program_id(0)
    a = agg_ref[...]
    ssum = jnp.sum(a, axis=1)
    ssq = jnp.sum(a * a, axis=1)

    @pl.when(i == 0)
    def _():
        out_ref[0] = ssum
        out_ref[1] = ssq

    @pl.when(i > 0)
    def _():
        out_ref[0] += ssum
        out_ref[1] += ssq


def _stats(agg):
    return pl.pallas_call(
        _stats_body,
        grid=(NB_N,),
        in_specs=[pl.BlockSpec((NT, BN, CW), lambda i: (0, i, 0))],
        out_specs=pl.BlockSpec((2, NT, CW), lambda i: (0, 0, 0)),
        out_shape=jax.ShapeDtypeStruct((2, NT, CW), jnp.float32),
    )(agg)


def _bn_from_stats(stats_ref, agg_blk):
    ssum = stats_ref[0].reshape(1, H)
    ssq = stats_ref[1].reshape(1, H)
    mu = ssum * jnp.float32(1.0 / N)
    var = ssq * jnp.float32(1.0 / N) - mu * mu
    inv = lax.rsqrt(var + jnp.float32(1e-5))
    aggf = jnp.concatenate([agg_blk[t] for t in range(NT)], axis=1)
    return (aggf - mu) * inv


def _applyt_body(agg_ref, h_ref, st_ref, gam_ref, bet_ref, wt1_ref, wt2_ref,
                 hn_ref, t1_ref, t2_ref):
    norm = _bn_from_stats(st_ref, agg_ref[...])
    hn = h_ref[...] + gam_ref[...] * norm + bet_ref[...]
    hn_ref[...] = hn
    for t in range(NT):
        t1_ref[t] = jnp.dot(hn, wt1_ref[t], preferred_element_type=jnp.float32,
                        precision=lax.Precision.HIGHEST)
        t2_ref[t] = jnp.dot(hn, wt2_ref[t], preferred_element_type=jnp.float32,
                        precision=lax.Precision.HIGHEST)


def _apply_tables(agg, h, st, gam, bet, wt1, wt2):
    return pl.pallas_call(
        _applyt_body,
        grid=(NB_N,),
        in_specs=[
            pl.BlockSpec((NT, BN, CW), lambda i: (0, i, 0)),
            pl.BlockSpec((BN, H), lambda i: (i, 0)),
            pl.BlockSpec((2, NT, CW), lambda i: (0, 0, 0)),
            pl.BlockSpec((1, H), lambda i: (0, 0)),
            pl.BlockSpec((1, H), lambda i: (0, 0)),
            pl.BlockSpec((NT, H, TW), lambda i: (0, 0, 0)),
            pl.BlockSpec((NT, H, TW), lambda i: (0, 0, 0)),
        ],
        out_specs=[
            pl.BlockSpec((BN, H), lambda i: (i, 0)),
            pl.BlockSpec((NT, BN, TW), lambda i: (0, i, 0)),
            pl.BlockSpec((NT, BN, TW), lambda i: (0, i, 0)),
        ],
        out_shape=[
            jax.ShapeDtypeStruct((N, H), jnp.float32),
            jax.ShapeDtypeStruct((NT, N, TW), jnp.float32),
            jax.ShapeDtypeStruct((NT, N, TW), jnp.float32),
        ],
    )(agg, h, st, gam, bet, wt1, wt2)


def _applyl_body(agg_ref, h_ref, st_ref, gam_ref, bet_ref, hn_ref):
    norm = _bn_from_stats(st_ref, agg_ref[...])
    hn_ref[...] = h_ref[...] + gam_ref[...] * norm + bet_ref[...]


def _apply_last(agg, h, st, gam, bet):
    return pl.pallas_call(
        _applyl_body,
        grid=(NB_N,),
        in_specs=[
            pl.BlockSpec((NT, BN, CW), lambda i: (0, i, 0)),
            pl.BlockSpec((BN, H), lambda i: (i, 0)),
            pl.BlockSpec((2, NT, CW), lambda i: (0, 0, 0)),
            pl.BlockSpec((1, H), lambda i: (0, 0)),
            pl.BlockSpec((1, H), lambda i: (0, 0)),
        ],
        out_specs=pl.BlockSpec((BN, H), lambda i: (i, 0)),
        out_shape=jax.ShapeDtypeStruct((N, H), jnp.float32),
    )(agg, h, st, gam, bet)


def _pool_body(h_ref, b_ref, sums_ref, cnt_ref):
    i = pl.program_id(0)
    hblk = h_ref[...]
    ids = b_ref[0]
    gids = lax.broadcasted_iota(jnp.int32, (G, BN), 0)
    oh = (gids == ids).astype(jnp.float32)
    s = jnp.dot(oh, hblk, preferred_element_type=jnp.float32,
                        precision=lax.Precision.HIGHEST)
    c = jnp.sum(oh, axis=1, keepdims=True)

    @pl.when(i == 0)
    def _():
        sums_ref[...] = s
        cnt_ref[...] = c

    @pl.when(i > 0)
    def _():
        sums_ref[...] += s
        cnt_ref[...] += c


def _pool(h, batch3):
    return pl.pallas_call(
        _pool_body,
        grid=(NB_N,),
        in_specs=[
            pl.BlockSpec((BN, H), lambda i: (i, 0)),
            pl.BlockSpec((1, 1, BN), lambda i: (i, 0, 0)),
        ],
        out_specs=[
            pl.BlockSpec((G, H), lambda i: (0, 0)),
            pl.BlockSpec((G, 1), lambda i: (0, 0)),
        ],
        out_shape=[
            jax.ShapeDtypeStruct((G, H), jnp.float32),
            jax.ShapeDtypeStruct((G, 1), jnp.float32),
        ],
    )(h, batch3)


def _head_body(s_ref, c_ref, w1_ref, b1_ref, w2_ref, b2_ref, w3_ref, b3_ref,
               out_ref):
    pooled = s_ref[...] / jnp.maximum(c_ref[...], 1.0)
    y = _tc_softplus(
        jnp.dot(pooled, w1_ref[...], preferred_element_type=jnp.float32,
                        precision=lax.Precision.HIGHEST)
        + b1_ref[...])
    y = _tc_softplus(
        jnp.dot(y, w2_ref[...], preferred_element_type=jnp.float32,
                        precision=lax.Precision.HIGHEST)
        + b2_ref[...])
    out_ref[...] = (
        jnp.dot(y, w3_ref[...], preferred_element_type=jnp.float32,
                        precision=lax.Precision.HIGHEST)
        + b3_ref[...])


def _head(sums, cnt, w1, b1, w2, b2, w3, b3):
    return pl.pallas_call(
        _head_body,
        out_shape=jax.ShapeDtypeStruct((G, 1), jnp.float32),
    )(sums, cnt, w1, b1, w2, b2, w3, b3)


def _edge_body(l, t1_hbm, t2_hbm, c_hbm, dst_hbm, src_hbm, z_hbm, out_hbm,
               idxd, idxs, idxa1, idxa2, g1, g2, cb, mb, accum,
               sem1, sem2, sem3):
    c = lax.axis_index("c")
    s = lax.axis_index("s")
    rows0 = s * ROWS_PER_TILE

    for p in range(2):
        for k in range(ROWS_PER_TILE // ZCH):
            pltpu.sync_copy(z_hbm, accum.at[pl.ds(rows0 + k * ZCH, ZCH)])
        plsc.subcore_barrier()

        tN = (p * 2) * N + c * N

        def chunk_body(j, carry):
            ch = s + NSUB * j

            @pl.when(ch < NCH)
            def _():
                off = ch * K
                pltpu.sync_copy(dst_hbm.at[pl.ds(off, K)], idxd)
                pltpu.sync_copy(src_hbm.at[pl.ds(off, K)], idxs)

                @plsc.parallel_loop(0, K, 16, unroll=4)
                def _adj(i):
                    idxa1[pl.ds(i, 16)] = idxd[pl.ds(i, 16)] + tN
                    idxa2[pl.ds(i, 16)] = idxs[pl.ds(i, 16)] + tN

                cp1 = pltpu.async_copy(t1_hbm.at[idxa1], g1, sem1)
                cp2 = pltpu.async_copy(t2_hbm.at[idxa2], g2, sem2)
                cp3 = pltpu.async_copy(
                    c_hbm.at[c, pl.ds(off, K), pl.ds((l * 2 + p) * TW, TW)],
                    cb, sem3)
                cp1.wait()
                cp2.wait()
                cp3.wait()

                @plsc.parallel_loop(0, K, 1, unroll=8)
                def _edge(e):
                    pf = (g1[e, pl.ds(0, CW)] + g2[e, pl.ds(0, CW)]
                          + cb[e, pl.ds(0, CW)])
                    ps = (g1[e, pl.ds(CW, CW)] + g2[e, pl.ds(CW, CW)]
                          + cb[e, pl.ds(CW, CW)])
                    mb[e, pl.ds(0, CW)] = _sc_sigmoid(pf) * _sc_softplus(ps)
                pltpu.sync_copy(mb, accum.at[idxd], add=True)

            return carry

        lax.fori_loop(0, MAXJ, chunk_body, 0)
        plsc.subcore_barrier()
        pltpu.sync_copy(accum.at[pl.ds(rows0, ROWS_PER_TILE)],
                        out_hbm.at[p, c, s])


@functools.lru_cache(maxsize=None)
def _make_edge_kernel(l):
    return pl.kernel(
        functools.partial(_edge_body, l),
        mesh=plsc.VectorSubcoreMesh(core_axis_name="c", subcore_axis_name="s"),
        compiler_params=pltpu.CompilerParams(use_tc_tiling_on_sc=False),
        out_type=jax.ShapeDtypeStruct((2, 2, NSUB, ROWS_PER_TILE, CW),
                                      jnp.float32),
        scratch_types=[
            pltpu.VMEM((K,), jnp.int32),
            pltpu.VMEM((K,), jnp.int32),
            pltpu.VMEM((K,), jnp.int32),
            pltpu.VMEM((K,), jnp.int32),
            pltpu.VMEM((K, TW), jnp.float32),
            pltpu.VMEM((K, TW), jnp.float32),
            pltpu.VMEM((K, TW), jnp.float32),
            pltpu.VMEM((K, CW), jnp.float32),
            pltpu.VMEM_SHARED((N, CW), jnp.float32),
            pltpu.SemaphoreType.DMA,
            pltpu.SemaphoreType.DMA,
            pltpu.SemaphoreType.DMA,
        ],
    )


def _edge_stage(t1, t2, c_all, dst, src, zeros, l):
    out = _make_edge_kernel(l)(t1, t2, c_all, dst, src, zeros)
    return out.reshape(NT, N, CW)


def kernel(x, edge_index, edge_attr, batch, W_emb, b_emb, Wf, bf, Ws, bs,
           gamma, beta, W1, b1, W2, b2, W3, b3):
    f32 = jnp.float32
    src = edge_index[0].astype(jnp.int32)
    dst = edge_index[1].astype(jnp.int32)

    wt1, wt2, wc, bc = [], [], [], []
    for l in range(L):
        w_f, w_s = Wf[l], Ws[l]
        q = lambda w, r0, t: w[r0:r0 + H, t * CW:(t + 1) * CW]
        wt1.append(jnp.stack([
            jnp.concatenate([q(w_f, 0, t), q(w_s, 0, t)], axis=1)
            for t in range(NT)]))
        wt2.append(jnp.stack([
            jnp.concatenate([q(w_f, H, t), q(w_s, H, t)], axis=1)
            for t in range(NT)]))
        wc.append(jnp.stack([
            jnp.concatenate([w_f[2 * H:, t * CW:(t + 1) * CW],
                             w_s[2 * H:, t * CW:(t + 1) * CW]], axis=1)
            for t in range(NT)]))
        bc.append(jnp.stack([
            jnp.concatenate([bf[l][t * CW:(t + 1) * CW],
                             bs[l][t * CW:(t + 1) * CW]])
            for t in range(NT)])[:, None, :])
    wt1 = jnp.stack(wt1)
    wt2 = jnp.stack(wt2)
    wc = jnp.stack(wc)
    bc = jnp.stack(bc)
    wcat = jnp.concatenate(
        [wc[l, p * 2 + c2] for c2 in range(2) for l in range(L)
         for p in range(2)], axis=1)
    bcat = jnp.concatenate(
        [bc[l, p * 2 + c2] for c2 in range(2) for l in range(L)
         for p in range(2)], axis=1)

    c_all = _cprep(edge_attr.astype(f32), wcat, bcat)
    zeros = jnp.zeros((ZCH, CW), f32)
    batch3 = batch.astype(jnp.int32).reshape(NB_N, 1, BN)

    h, t1, t2 = _embed(x.astype(f32), W_emb, b_emb.reshape(1, H),
                       wt1[0], wt2[0])
    for l in range(L):
        agg = _edge_stage(t1.reshape(NT * N, TW), t2.reshape(NT * N, TW),
                          c_all, dst, src, zeros, l)
        st = _stats(agg)
        gam = gamma[l].reshape(1, H)
        bet = beta[l].reshape(1, H)
        if l < L - 1:
            h, t1, t2 = _apply_tables(agg, h, st, gam, bet,
                                      wt1[l + 1], wt2[l + 1])
        else:
            h = _apply_last(agg, h, st, gam, bet)

    sums, cnt = _pool(h, batch3)
    return _head(sums, cnt, W1, b1.reshape(1, 128), W2, b2.reshape(1, 64),
                 W3, b3.reshape(1, 1))


# speedup vs baseline: 3.1389x; 1.1612x over previous
import functools

import jax
import jax.numpy as jnp
from jax import lax
from jax.experimental import pallas as pl
from jax.experimental.pallas import tpu as pltpu
from jax.experimental.pallas import tpu_sc as plsc

N = 50000
E = 800000
FEA = 92
H = 64
EDIM = 41
L = 4
G = 256
CW = 16
TW = 2 * CW
NT = 4

K = 256
NCH = E // K
NSUB = 16
ROWS_PER_TILE = N // NSUB
ZCH = 625
MAXJ = (NCH + NSUB - 1) // NSUB

BE = 1000
BN = 400
NB_N = N // BN

_LOG1P_C = (
    0.9999751958009936, -0.4993836513699655, 0.32778470930088355,
    -0.22478407188994595, 0.13329712681080494, -0.0543128720320267,
    0.010571220016300703,
)


def _poly_log1p(u):
    r = jnp.float32(_LOG1P_C[-1])
    for c in _LOG1P_C[-2::-1]:
        r = r * u + jnp.float32(c)
    return r * u


_EXP_C = (1.0 / 720, 1.0 / 120, 1.0 / 24, 1.0 / 6, 0.5, 1.0, 1.0)


def _sc_exp(x):
    t = x * jnp.float32(1.4426950408889634)
    t = jnp.minimum(jnp.maximum(t, jnp.float32(-126.0)), jnp.float32(126.0))
    half = jnp.where(t >= 0, jnp.float32(0.5), jnp.float32(-0.5))
    n = (t + half).astype(jnp.int32)
    y = (t - n.astype(jnp.float32)) * jnp.float32(0.6931471805599453)
    p = jnp.float32(_EXP_C[0])
    for c in _EXP_C[1:]:
        p = p * y + jnp.float32(c)
    scale = lax.bitcast_convert_type((n + 127) << 23, jnp.float32)
    return p * scale


def _sc_rcp(d):
    r = lax.bitcast_convert_type(
        jnp.int32(0x7EF311C3) - lax.bitcast_convert_type(d, jnp.int32),
        jnp.float32)
    for _ in range(3):
        r = r * (jnp.float32(2.0) - d * r)
    return r


def _sc_softplus(x):
    u = _sc_exp(-jnp.abs(x))
    return jnp.maximum(x, jnp.float32(0.0)) + _poly_log1p(u)


def _sc_sigmoid(x):
    return _sc_rcp(jnp.float32(1.0) + _sc_exp(-x))


def _tc_softplus(x):
    return jnp.maximum(x, 0.0) + jnp.log1p(jnp.exp(-jnp.abs(x)))


_CJ = 2 * L * TW


def _cprep_body(ea_ref, w_ref, b_ref, out_ref):
    r = jnp.dot(ea_ref[...], w_ref[...], preferred_element_type=jnp.float32,
                precision=lax.Precision.HIGHEST) + b_ref[...]
    out_ref[0] = r[:, :_CJ]
    out_ref[1] = r[:, _CJ:]


def _cprep(ea, wcat, bcat):
    return pl.pallas_call(
        _cprep_body,
        grid=(E // BE,),
        in_specs=[
            pl.BlockSpec((BE, EDIM), lambda i: (i, 0)),
            pl.BlockSpec((EDIM, 2 * _CJ), lambda i: (0, 0)),
            pl.BlockSpec((1, 2 * _CJ), lambda i: (0, 0)),
        ],
        out_specs=pl.BlockSpec((2, BE, _CJ), lambda i: (0, i, 0)),
        out_shape=jax.ShapeDtypeStruct((2, E, _CJ), jnp.float32),
    )(ea, wcat, bcat)


def _embed_body(x_ref, we_ref, be_ref, wt1_ref, wt2_ref, h_ref, t1_ref, t2_ref):
    h = jnp.dot(x_ref[...], we_ref[...], preferred_element_type=jnp.float32,
                        precision=lax.Precision.HIGHEST)
    h = h + be_ref[...]
    h_ref[...] = h
    for t in range(NT):
        t1_ref[t] = jnp.dot(h, wt1_ref[t], preferred_element_type=jnp.float32,
                        precision=lax.Precision.HIGHEST)
        t2_ref[t] = jnp.dot(h, wt2_ref[t], preferred_element_type=jnp.float32,
                        precision=lax.Precision.HIGHEST)


def _embed(x, we, be, wt1, wt2):
    return pl.pallas_call(
        _embed_body,
        grid=(NB_N,),
        in_specs=[
            pl.BlockSpec((BN, FEA), lambda i: (i, 0)),
            pl.BlockSpec((FEA, H), lambda i: (0, 0)),
            pl.BlockSpec((1, H), lambda i: (0, 0)),
            pl.BlockSpec((NT, H, TW), lambda i: (0, 0, 0)),
            pl.BlockSpec((NT, H, TW), lambda i: (0, 0, 0)),
        ],
        out_specs=[
            pl.BlockSpec((BN, H), lambda i: (i, 0)),
            pl.BlockSpec((NT, BN, TW), lambda i: (0, i, 0)),
            pl.BlockSpec((NT, BN, TW), lambda i: (0, i, 0)),
        ],
        out_shape=[
            jax.ShapeDtypeStruct((N, H), jnp.float32),
            jax.ShapeDtypeStruct((NT, N, TW), jnp.float32),
            jax.ShapeDtypeStruct((NT, N, TW), jnp.float32),
        ],
    )(x, we, be, wt1, wt2)


def _stats_body(agg_ref, out_ref):
    i = pl.program_id(0)
    a = agg_ref[...]
    ssum = jnp.sum(a, axis=1)
    ssq = jnp.sum(a * a, axis=1)

    @pl.when(i == 0)
    def _():
        out_ref[0] = ssum
        out_ref[1] = ssq

    @pl.when(i > 0)
    def _():
        out_ref[0] += ssum
        out_ref[1] += ssq


def _stats(agg):
    return pl.pallas_call(
        _stats_body,
        grid=(NB_N,),
        in_specs=[pl.BlockSpec((NT, BN, CW), lambda i: (0, i, 0))],
        out_specs=pl.BlockSpec((2, NT, CW), lambda i: (0, 0, 0)),
        out_shape=jax.ShapeDtypeStruct((2, NT, CW), jnp.float32),
    )(agg)


def _bn_from_stats(stats_ref, agg_blk):
    ssum = stats_ref[0].reshape(1, H)
    ssq = stats_ref[1].reshape(1, H)
    mu = ssum * jnp.float32(1.0 / N)
    var = ssq * jnp.float32(1.0 / N) - mu * mu
    inv = lax.rsqrt(var + jnp.float32(1e-5))
    aggf = jnp.concatenate([agg_blk[t] for t in range(NT)], axis=1)
    return (aggf - mu) * inv


def _applyt_body(agg_ref, h_ref, st_ref, gam_ref, bet_ref, wt1_ref, wt2_ref,
                 hn_ref, t1_ref, t2_ref):
    norm = _bn_from_stats(st_ref, agg_ref[...])
    hn = h_ref[...] + gam_ref[...] * norm + bet_ref[...]
    hn_ref[...] = hn
    for t in range(NT):
        t1_ref[t] = jnp.dot(hn, wt1_ref[t], preferred_element_type=jnp.float32,
                        precision=lax.Precision.HIGHEST)
        t2_ref[t] = jnp.dot(hn, wt2_ref[t], preferred_element_type=jnp.float32,
                        precision=lax.Precision.HIGHEST)


def _apply_tables(agg, h, st, gam, bet, wt1, wt2):
    return pl.pallas_call(
        _applyt_body,
        grid=(NB_N,),
        in_specs=[
            pl.BlockSpec((NT, BN, CW), lambda i: (0, i, 0)),
            pl.BlockSpec((BN, H), lambda i: (i, 0)),
            pl.BlockSpec((2, NT, CW), lambda i: (0, 0, 0)),
            pl.BlockSpec((1, H), lambda i: (0, 0)),
            pl.BlockSpec((1, H), lambda i: (0, 0)),
            pl.BlockSpec((NT, H, TW), lambda i: (0, 0, 0)),
            pl.BlockSpec((NT, H, TW), lambda i: (0, 0, 0)),
        ],
        out_specs=[
            pl.BlockSpec((BN, H), lambda i: (i, 0)),
            pl.BlockSpec((NT, BN, TW), lambda i: (0, i, 0)),
            pl.BlockSpec((NT, BN, TW), lambda i: (0, i, 0)),
        ],
        out_shape=[
            jax.ShapeDtypeStruct((N, H), jnp.float32),
            jax.ShapeDtypeStruct((NT, N, TW), jnp.float32),
            jax.ShapeDtypeStruct((NT, N, TW), jnp.float32),
        ],
    )(agg, h, st, gam, bet, wt1, wt2)


def _applyl_body(agg_ref, h_ref, st_ref, gam_ref, bet_ref, hn_ref):
    norm = _bn_from_stats(st_ref, agg_ref[...])
    hn_ref[...] = h_ref[...] + gam_ref[...] * norm + bet_ref[...]


def _apply_last(agg, h, st, gam, bet):
    return pl.pallas_call(
        _applyl_body,
        grid=(NB_N,),
        in_specs=[
            pl.BlockSpec((NT, BN, CW), lambda i: (0, i, 0)),
            pl.BlockSpec((BN, H), lambda i: (i, 0)),
            pl.BlockSpec((2, NT, CW), lambda i: (0, 0, 0)),
            pl.BlockSpec((1, H), lambda i: (0, 0)),
            pl.BlockSpec((1, H), lambda i: (0, 0)),
        ],
        out_specs=pl.BlockSpec((BN, H), lambda i: (i, 0)),
        out_shape=jax.ShapeDtypeStruct((N, H), jnp.float32),
    )(agg, h, st, gam, bet)


def _pool_body(h_ref, b_ref, sums_ref, cnt_ref):
    i = pl.program_id(0)
    hblk = h_ref[...]
    ids = b_ref[0]
    gids = lax.broadcasted_iota(jnp.int32, (G, BN), 0)
    oh = (gids == ids).astype(jnp.float32)
    s = jnp.dot(oh, hblk, preferred_element_type=jnp.float32,
                        precision=lax.Precision.HIGHEST)
    c = jnp.sum(oh, axis=1, keepdims=True)

    @pl.when(i == 0)
    def _():
        sums_ref[...] = s
        cnt_ref[...] = c

    @pl.when(i > 0)
    def _():
        sums_ref[...] += s
        cnt_ref[...] += c


def _pool(h, batch3):
    return pl.pallas_call(
        _pool_body,
        grid=(NB_N,),
        in_specs=[
            pl.BlockSpec((BN, H), lambda i: (i, 0)),
            pl.BlockSpec((1, 1, BN), lambda i: (i, 0, 0)),
        ],
        out_specs=[
            pl.BlockSpec((G, H), lambda i: (0, 0)),
            pl.BlockSpec((G, 1), lambda i: (0, 0)),
        ],
        out_shape=[
            jax.ShapeDtypeStruct((G, H), jnp.float32),
            jax.ShapeDtypeStruct((G, 1), jnp.float32),
        ],
    )(h, batch3)


def _head_body(s_ref, c_ref, w1_ref, b1_ref, w2_ref, b2_ref, w3_ref, b3_ref,
               out_ref):
    pooled = s_ref[...] / jnp.maximum(c_ref[...], 1.0)
    y = _tc_softplus(
        jnp.dot(pooled, w1_ref[...], preferred_element_type=jnp.float32,
                        precision=lax.Precision.HIGHEST)
        + b1_ref[...])
    y = _tc_softplus(
        jnp.dot(y, w2_ref[...], preferred_element_type=jnp.float32,
                        precision=lax.Precision.HIGHEST)
        + b2_ref[...])
    out_ref[...] = (
        jnp.dot(y, w3_ref[...], preferred_element_type=jnp.float32,
                        precision=lax.Precision.HIGHEST)
        + b3_ref[...])


def _head(sums, cnt, w1, b1, w2, b2, w3, b3):
    return pl.pallas_call(
        _head_body,
        out_shape=jax.ShapeDtypeStruct((G, 1), jnp.float32),
    )(sums, cnt, w1, b1, w2, b2, w3, b3)


def _edge_body(l, t1_hbm, t2_hbm, c_hbm, dst_hbm, src_hbm, z_hbm, out_hbm,
               idxd, idxs, idxa1, idxa2, g1, g2, cb, mb, accum, semg):
    c = lax.axis_index("c")
    s = lax.axis_index("s")
    rows0 = s * ROWS_PER_TILE

    for p in range(2):
        for k in range(ROWS_PER_TILE // ZCH):
            pltpu.sync_copy(z_hbm, accum.at[pl.ds(rows0 + k * ZCH, ZCH)])
        plsc.subcore_barrier()

        tN = (p * 2) * N + c * N
        col0 = (l * 2 + p) * TW

        def issue(j, b):
            ch = s + NSUB * j

            @pl.when(ch < NCH)
            def _():
                off = ch * K
                pltpu.sync_copy(dst_hbm.at[pl.ds(off, K)], idxd[b])
                pltpu.sync_copy(src_hbm.at[pl.ds(off, K)], idxs)

                @plsc.parallel_loop(0, K, 16, unroll=4)
                def _adj(i):
                    idxa1[b][pl.ds(i, 16)] = idxd[b][pl.ds(i, 16)] + tN
                    idxa2[b][pl.ds(i, 16)] = idxs[pl.ds(i, 16)] + tN

                pltpu.async_copy(t1_hbm.at[idxa1[b]], g1[b], semg[b])
                pltpu.async_copy(t2_hbm.at[idxa2[b]], g2[b], semg[b])
                pltpu.async_copy(
                    c_hbm.at[c, pl.ds(off, K), pl.ds(col0, TW)],
                    cb[b], semg[b])

        issue(jnp.int32(0), 0)
        issue(jnp.int32(1), 1)

        def outer(jj, carry):
            for b in range(2):
                j = jj * 2 + b
                ch = s + NSUB * j

                @pl.when(ch < NCH)
                def _():
                    pltpu.make_async_copy(
                        t1_hbm.at[idxa1[b]], g1[b], semg[b]).wait()
                    pltpu.make_async_copy(
                        t2_hbm.at[idxa2[b]], g2[b], semg[b]).wait()
                    pltpu.make_async_copy(
                        c_hbm.at[c, pl.ds(ch * K, K), pl.ds(col0, TW)],
                        cb[b], semg[b]).wait()

                    @plsc.parallel_loop(0, K, 1, unroll=8)
                    def _edge(e):
                        pf = (g1[b][e, pl.ds(0, CW)] + g2[b][e, pl.ds(0, CW)]
                              + cb[b][e, pl.ds(0, CW)])
                        ps = (g1[b][e, pl.ds(CW, CW)]
                              + g2[b][e, pl.ds(CW, CW)]
                              + cb[b][e, pl.ds(CW, CW)])
                        mb[e, pl.ds(0, CW)] = _sc_sigmoid(pf) * _sc_softplus(ps)

                    pltpu.sync_copy(mb, accum.at[idxd[b]], add=True)
                    issue(j + 2, b)

            return carry

        lax.fori_loop(0, MAXJ // 2, outer, 0)
        plsc.subcore_barrier()
        pltpu.sync_copy(accum.at[pl.ds(rows0, ROWS_PER_TILE)],
                        out_hbm.at[p, c, s])


@functools.lru_cache(maxsize=None)
def _make_edge_kernel(l):
    return pl.kernel(
        functools.partial(_edge_body, l),
        mesh=plsc.VectorSubcoreMesh(core_axis_name="c", subcore_axis_name="s"),
        compiler_params=pltpu.CompilerParams(use_tc_tiling_on_sc=False),
        out_type=jax.ShapeDtypeStruct((2, 2, NSUB, ROWS_PER_TILE, CW),
                                      jnp.float32),
        scratch_types=[
            [pltpu.VMEM((K,), jnp.int32)] * 2,
            pltpu.VMEM((K,), jnp.int32),
            [pltpu.VMEM((K,), jnp.int32)] * 2,
            [pltpu.VMEM((K,), jnp.int32)] * 2,
            [pltpu.VMEM((K, TW), jnp.float32)] * 2,
            [pltpu.VMEM((K, TW), jnp.float32)] * 2,
            [pltpu.VMEM((K, TW), jnp.float32)] * 2,
            pltpu.VMEM((K, CW), jnp.float32),
            pltpu.VMEM_SHARED((N, CW), jnp.float32),
            [pltpu.SemaphoreType.DMA] * 2,
        ],
    )


def _edge_stage(t1, t2, c_all, dst, src, zeros, l):
    out = _make_edge_kernel(l)(t1, t2, c_all, dst, src, zeros)
    return out.reshape(NT, N, CW)


def kernel(x, edge_index, edge_attr, batch, W_emb, b_emb, Wf, bf, Ws, bs,
           gamma, beta, W1, b1, W2, b2, W3, b3):
    f32 = jnp.float32
    src = edge_index[0].astype(jnp.int32)
    dst = edge_index[1].astype(jnp.int32)

    wt1, wt2, wc, bc = [], [], [], []
    for l in range(L):
        w_f, w_s = Wf[l], Ws[l]
        q = lambda w, r0, t: w[r0:r0 + H, t * CW:(t + 1) * CW]
        wt1.append(jnp.stack([
            jnp.concatenate([q(w_f, 0, t), q(w_s, 0, t)], axis=1)
            for t in range(NT)]))
        wt2.append(jnp.stack([
            jnp.concatenate([q(w_f, H, t), q(w_s, H, t)], axis=1)
            for t in range(NT)]))
        wc.append(jnp.stack([
            jnp.concatenate([w_f[2 * H:, t * CW:(t + 1) * CW],
                             w_s[2 * H:, t * CW:(t + 1) * CW]], axis=1)
            for t in range(NT)]))
        bc.append(jnp.stack([
            jnp.concatenate([bf[l][t * CW:(t + 1) * CW],
                             bs[l][t * CW:(t + 1) * CW]])
            for t in range(NT)])[:, None, :])
    wt1 = jnp.stack(wt1)
    wt2 = jnp.stack(wt2)
    wc = jnp.stack(wc)
    bc = jnp.stack(bc)
    wcat = jnp.concatenate(
        [wc[l, p * 2 + c2] for c2 in range(2) for l in range(L)
         for p in range(2)], axis=1)
    bcat = jnp.concatenate(
        [bc[l, p * 2 + c2] for c2 in range(2) for l in range(L)
         for p in range(2)], axis=1)

    c_all = _cprep(edge_attr.astype(f32), wcat, bcat)
    zeros = jnp.zeros((ZCH, CW), f32)
    batch3 = batch.astype(jnp.int32).reshape(NB_N, 1, BN)

    h, t1, t2 = _embed(x.astype(f32), W_emb, b_emb.reshape(1, H),
                       wt1[0], wt2[0])
    for l in range(L):
        agg = _edge_stage(t1.reshape(NT * N, TW), t2.reshape(NT * N, TW),
                          c_all, dst, src, zeros, l)
        st = _stats(agg)
        gam = gamma[l].reshape(1, H)
        bet = beta[l].reshape(1, H)
        if l < L - 1:
            h, t1, t2 = _apply_tables(agg, h, st, gam, bet,
                                      wt1[l + 1], wt2[l + 1])
        else:
            h = _apply_last(agg, h, st, gam, bet)

    sums, cnt = _pool(h, batch3)
    return _head(sums, cnt, W1, b1.reshape(1, 128), W2, b2.reshape(1, 64),
                 W3, b3.reshape(1, 1))


# speedup vs baseline: 4.2781x; 1.3630x over previous
import functools

import jax
import jax.numpy as jnp
from jax import lax
from jax.experimental import pallas as pl
from jax.experimental.pallas import tpu as pltpu
from jax.experimental.pallas import tpu_sc as plsc

N = 50000
E = 800000
FEA = 92
H = 64
EDIM = 41
L = 4
G = 256
CW = 16
TW = 2 * CW
NT = 4

K = 256
NCH = E // K
NSUB = 16
ROWS_PER_TILE = N // NSUB
ZCH = 625
MAXJ = (NCH + NSUB - 1) // NSUB

BE = 1000
BN = 400
NB_N = N // BN

_LOG1P_C = (
    0.9999751958009936, -0.4993836513699655, 0.32778470930088355,
    -0.22478407188994595, 0.13329712681080494, -0.0543128720320267,
    0.010571220016300703,
)


def _poly_log1p(u):
    r = jnp.float32(_LOG1P_C[-1])
    for c in _LOG1P_C[-2::-1]:
        r = r * u + jnp.float32(c)
    return r * u


_EXP_C = (1.0 / 720, 1.0 / 120, 1.0 / 24, 1.0 / 6, 0.5, 1.0, 1.0)


def _sc_exp(x):
    t = x * jnp.float32(1.4426950408889634)
    t = jnp.minimum(jnp.maximum(t, jnp.float32(-126.0)), jnp.float32(126.0))
    half = jnp.where(t >= 0, jnp.float32(0.5), jnp.float32(-0.5))
    n = (t + half).astype(jnp.int32)
    y = (t - n.astype(jnp.float32)) * jnp.float32(0.6931471805599453)
    p = jnp.float32(_EXP_C[0])
    for c in _EXP_C[1:]:
        p = p * y + jnp.float32(c)
    scale = lax.bitcast_convert_type((n + 127) << 23, jnp.float32)
    return p * scale


def _sc_rcp(d):
    r = lax.bitcast_convert_type(
        jnp.int32(0x7EF311C3) - lax.bitcast_convert_type(d, jnp.int32),
        jnp.float32)
    for _ in range(3):
        r = r * (jnp.float32(2.0) - d * r)
    return r


def _sc_softplus(x):
    u = jnp.exp(-jnp.abs(x))
    return jnp.maximum(x, jnp.float32(0.0)) + _poly_log1p(u)


def _sc_sigmoid(x):
    return _sc_rcp(jnp.float32(1.0) + jnp.exp(-x))


def _tc_softplus(x):
    return jnp.maximum(x, 0.0) + jnp.log1p(jnp.exp(-jnp.abs(x)))


_CJ = 2 * L * TW


def _cprep_body(ea_ref, w_ref, b_ref, out_ref):
    r = jnp.dot(ea_ref[...], w_ref[...], preferred_element_type=jnp.float32,
                precision=lax.Precision.HIGHEST) + b_ref[...]
    out_ref[0] = r[:, :_CJ]
    out_ref[1] = r[:, _CJ:]


def _cprep(ea, wcat, bcat):
    return pl.pallas_call(
        _cprep_body,
        grid=(E // BE,),
        in_specs=[
            pl.BlockSpec((BE, EDIM), lambda i: (i, 0)),
            pl.BlockSpec((EDIM, 2 * _CJ), lambda i: (0, 0)),
            pl.BlockSpec((1, 2 * _CJ), lambda i: (0, 0)),
        ],
        out_specs=pl.BlockSpec((2, BE, _CJ), lambda i: (0, i, 0)),
        out_shape=jax.ShapeDtypeStruct((2, E, _CJ), jnp.float32),
    )(ea, wcat, bcat)


def _embed_body(x_ref, we_ref, be_ref, wt1_ref, wt2_ref, h_ref, t1_ref, t2_ref):
    h = jnp.dot(x_ref[...], we_ref[...], preferred_element_type=jnp.float32,
                        precision=lax.Precision.HIGHEST)
    h = h + be_ref[...]
    h_ref[...] = h
    for t in range(NT):
        t1_ref[t] = jnp.dot(h, wt1_ref[t], preferred_element_type=jnp.float32,
                        precision=lax.Precision.HIGHEST)
        t2_ref[t] = jnp.dot(h, wt2_ref[t], preferred_element_type=jnp.float32,
                        precision=lax.Precision.HIGHEST)


def _embed(x, we, be, wt1, wt2):
    return pl.pallas_call(
        _embed_body,
        grid=(NB_N,),
        in_specs=[
            pl.BlockSpec((BN, FEA), lambda i: (i, 0)),
            pl.BlockSpec((FEA, H), lambda i: (0, 0)),
            pl.BlockSpec((1, H), lambda i: (0, 0)),
            pl.BlockSpec((NT, H, TW), lambda i: (0, 0, 0)),
            pl.BlockSpec((NT, H, TW), lambda i: (0, 0, 0)),
        ],
        out_specs=[
            pl.BlockSpec((BN, H), lambda i: (i, 0)),
            pl.BlockSpec((NT, BN, TW), lambda i: (0, i, 0)),
            pl.BlockSpec((NT, BN, TW), lambda i: (0, i, 0)),
        ],
        out_shape=[
            jax.ShapeDtypeStruct((N, H), jnp.float32),
            jax.ShapeDtypeStruct((NT, N, TW), jnp.float32),
            jax.ShapeDtypeStruct((NT, N, TW), jnp.float32),
        ],
    )(x, we, be, wt1, wt2)


def _stats_body(agg_ref, out_ref):
    i = pl.program_id(0)
    a = agg_ref[...]
    ssum = jnp.sum(a, axis=1)
    ssq = jnp.sum(a * a, axis=1)

    @pl.when(i == 0)
    def _():
        out_ref[0] = ssum
        out_ref[1] = ssq

    @pl.when(i > 0)
    def _():
        out_ref[0] += ssum
        out_ref[1] += ssq


def _stats(agg):
    return pl.pallas_call(
        _stats_body,
        grid=(NB_N,),
        in_specs=[pl.BlockSpec((NT, BN, CW), lambda i: (0, i, 0))],
        out_specs=pl.BlockSpec((2, NT, CW), lambda i: (0, 0, 0)),
        out_shape=jax.ShapeDtypeStruct((2, NT, CW), jnp.float32),
    )(agg)


def _bn_from_stats(stats_ref, agg_blk):
    ssum = stats_ref[0].reshape(1, H)
    ssq = stats_ref[1].reshape(1, H)
    mu = ssum * jnp.float32(1.0 / N)
    var = ssq * jnp.float32(1.0 / N) - mu * mu
    inv = lax.rsqrt(var + jnp.float32(1e-5))
    aggf = jnp.concatenate([agg_blk[t] for t in range(NT)], axis=1)
    return (aggf - mu) * inv


def _applyt_body(agg_ref, h_ref, st_ref, gam_ref, bet_ref, wt1_ref, wt2_ref,
                 hn_ref, t1_ref, t2_ref):
    norm = _bn_from_stats(st_ref, agg_ref[...])
    hn = h_ref[...] + gam_ref[...] * norm + bet_ref[...]
    hn_ref[...] = hn
    for t in range(NT):
        t1_ref[t] = jnp.dot(hn, wt1_ref[t], preferred_element_type=jnp.float32,
                        precision=lax.Precision.HIGHEST)
        t2_ref[t] = jnp.dot(hn, wt2_ref[t], preferred_element_type=jnp.float32,
                        precision=lax.Precision.HIGHEST)


def _apply_tables(agg, h, st, gam, bet, wt1, wt2):
    return pl.pallas_call(
        _applyt_body,
        grid=(NB_N,),
        in_specs=[
            pl.BlockSpec((NT, BN, CW), lambda i: (0, i, 0)),
            pl.BlockSpec((BN, H), lambda i: (i, 0)),
            pl.BlockSpec((2, NT, CW), lambda i: (0, 0, 0)),
            pl.BlockSpec((1, H), lambda i: (0, 0)),
            pl.BlockSpec((1, H), lambda i: (0, 0)),
            pl.BlockSpec((NT, H, TW), lambda i: (0, 0, 0)),
            pl.BlockSpec((NT, H, TW), lambda i: (0, 0, 0)),
        ],
        out_specs=[
            pl.BlockSpec((BN, H), lambda i: (i, 0)),
            pl.BlockSpec((NT, BN, TW), lambda i: (0, i, 0)),
            pl.BlockSpec((NT, BN, TW), lambda i: (0, i, 0)),
        ],
        out_shape=[
            jax.ShapeDtypeStruct((N, H), jnp.float32),
            jax.ShapeDtypeStruct((NT, N, TW), jnp.float32),
            jax.ShapeDtypeStruct((NT, N, TW), jnp.float32),
        ],
    )(agg, h, st, gam, bet, wt1, wt2)


def _applyl_body(agg_ref, h_ref, st_ref, gam_ref, bet_ref, hn_ref):
    norm = _bn_from_stats(st_ref, agg_ref[...])
    hn_ref[...] = h_ref[...] + gam_ref[...] * norm + bet_ref[...]


def _apply_last(agg, h, st, gam, bet):
    return pl.pallas_call(
        _applyl_body,
        grid=(NB_N,),
        in_specs=[
            pl.BlockSpec((NT, BN, CW), lambda i: (0, i, 0)),
            pl.BlockSpec((BN, H), lambda i: (i, 0)),
            pl.BlockSpec((2, NT, CW), lambda i: (0, 0, 0)),
            pl.BlockSpec((1, H), lambda i: (0, 0)),
            pl.BlockSpec((1, H), lambda i: (0, 0)),
        ],
        out_specs=pl.BlockSpec((BN, H), lambda i: (i, 0)),
        out_shape=jax.ShapeDtypeStruct((N, H), jnp.float32),
    )(agg, h, st, gam, bet)


def _pool_body(h_ref, b_ref, sums_ref, cnt_ref):
    i = pl.program_id(0)
    hblk = h_ref[...]
    ids = b_ref[0]
    gids = lax.broadcasted_iota(jnp.int32, (G, BN), 0)
    oh = (gids == ids).astype(jnp.float32)
    s = jnp.dot(oh, hblk, preferred_element_type=jnp.float32,
                        precision=lax.Precision.HIGHEST)
    c = jnp.sum(oh, axis=1, keepdims=True)

    @pl.when(i == 0)
    def _():
        sums_ref[...] = s
        cnt_ref[...] = c

    @pl.when(i > 0)
    def _():
        sums_ref[...] += s
        cnt_ref[...] += c


def _pool(h, batch3):
    return pl.pallas_call(
        _pool_body,
        grid=(NB_N,),
        in_specs=[
            pl.BlockSpec((BN, H), lambda i: (i, 0)),
            pl.BlockSpec((1, 1, BN), lambda i: (i, 0, 0)),
        ],
        out_specs=[
            pl.BlockSpec((G, H), lambda i: (0, 0)),
            pl.BlockSpec((G, 1), lambda i: (0, 0)),
        ],
        out_shape=[
            jax.ShapeDtypeStruct((G, H), jnp.float32),
            jax.ShapeDtypeStruct((G, 1), jnp.float32),
        ],
    )(h, batch3)


def _head_body(s_ref, c_ref, w1_ref, b1_ref, w2_ref, b2_ref, w3_ref, b3_ref,
               out_ref):
    pooled = s_ref[...] / jnp.maximum(c_ref[...], 1.0)
    y = _tc_softplus(
        jnp.dot(pooled, w1_ref[...], preferred_element_type=jnp.float32,
                        precision=lax.Precision.HIGHEST)
        + b1_ref[...])
    y = _tc_softplus(
        jnp.dot(y, w2_ref[...], preferred_element_type=jnp.float32,
                        precision=lax.Precision.HIGHEST)
        + b2_ref[...])
    out_ref[...] = (
        jnp.dot(y, w3_ref[...], preferred_element_type=jnp.float32,
                        precision=lax.Precision.HIGHEST)
        + b3_ref[...])


def _head(sums, cnt, w1, b1, w2, b2, w3, b3):
    return pl.pallas_call(
        _head_body,
        out_shape=jax.ShapeDtypeStruct((G, 1), jnp.float32),
    )(sums, cnt, w1, b1, w2, b2, w3, b3)


def _edge_body(l, t1_hbm, t2_hbm, c_hbm, dst_hbm, src_hbm, z_hbm, out_hbm,
               idxd, idxs, idxa1, idxa2, g1, g2, cb, mb, accum, semg):
    c = lax.axis_index("c")
    s = lax.axis_index("s")
    rows0 = s * ROWS_PER_TILE

    for p in range(2):
        for k in range(ROWS_PER_TILE // ZCH):
            pltpu.sync_copy(z_hbm, accum.at[pl.ds(rows0 + k * ZCH, ZCH)])
        plsc.subcore_barrier()

        tN = (p * 2) * N + c * N
        col0 = (l * 2 + p) * TW

        def issue(j, b):
            ch = s + NSUB * j

            @pl.when(ch < NCH)
            def _():
                off = ch * K
                pltpu.sync_copy(dst_hbm.at[pl.ds(off, K)], idxd[b])
                pltpu.sync_copy(src_hbm.at[pl.ds(off, K)], idxs)

                @plsc.parallel_loop(0, K, 16, unroll=4)
                def _adj(i):
                    idxa1[b][pl.ds(i, 16)] = idxd[b][pl.ds(i, 16)] + tN
                    idxa2[b][pl.ds(i, 16)] = idxs[pl.ds(i, 16)] + tN

                pltpu.async_copy(t1_hbm.at[idxa1[b]], g1[b], semg[b])
                pltpu.async_copy(t2_hbm.at[idxa2[b]], g2[b], semg[b])
                pltpu.async_copy(
                    c_hbm.at[c, pl.ds(off, K), pl.ds(col0, TW)],
                    cb[b], semg[b])

        issue(jnp.int32(0), 0)
        issue(jnp.int32(1), 1)

        def outer(jj, carry):
            for b in range(2):
                j = jj * 2 + b
                ch = s + NSUB * j

                @pl.when(ch < NCH)
                def _():
                    pltpu.make_async_copy(
                        t1_hbm.at[idxa1[b]], g1[b], semg[b]).wait()
                    pltpu.make_async_copy(
                        t2_hbm.at[idxa2[b]], g2[b], semg[b]).wait()
                    pltpu.make_async_copy(
                        c_hbm.at[c, pl.ds(ch * K, K), pl.ds(col0, TW)],
                        cb[b], semg[b]).wait()

                    @plsc.parallel_loop(0, K, 1, unroll=8)
                    def _edge(e):
                        pf = (g1[b][e, pl.ds(0, CW)] + g2[b][e, pl.ds(0, CW)]
                              + cb[b][e, pl.ds(0, CW)])
                        ps = (g1[b][e, pl.ds(CW, CW)]
                              + g2[b][e, pl.ds(CW, CW)]
                              + cb[b][e, pl.ds(CW, CW)])
                        mb[e, pl.ds(0, CW)] = _sc_sigmoid(pf) * _sc_softplus(ps)

                    pltpu.sync_copy(mb, accum.at[idxd[b]], add=True)
                    issue(j + 2, b)

            return carry

        lax.fori_loop(0, MAXJ // 2, outer, 0)
        plsc.subcore_barrier()
        pltpu.sync_copy(accum.at[pl.ds(rows0, ROWS_PER_TILE)],
                        out_hbm.at[p, c, s])


@functools.lru_cache(maxsize=None)
def _make_edge_kernel(l):
    return pl.kernel(
        functools.partial(_edge_body, l),
        mesh=plsc.VectorSubcoreMesh(core_axis_name="c", subcore_axis_name="s"),
        compiler_params=pltpu.CompilerParams(use_tc_tiling_on_sc=False),
        out_type=jax.ShapeDtypeStruct((2, 2, NSUB, ROWS_PER_TILE, CW),
                                      jnp.float32),
        scratch_types=[
            [pltpu.VMEM((K,), jnp.int32)] * 2,
            pltpu.VMEM((K,), jnp.int32),
            [pltpu.VMEM((K,), jnp.int32)] * 2,
            [pltpu.VMEM((K,), jnp.int32)] * 2,
            [pltpu.VMEM((K, TW), jnp.float32)] * 2,
            [pltpu.VMEM((K, TW), jnp.float32)] * 2,
            [pltpu.VMEM((K, TW), jnp.float32)] * 2,
            pltpu.VMEM((K, CW), jnp.float32),
            pltpu.VMEM_SHARED((N, CW), jnp.float32),
            [pltpu.SemaphoreType.DMA] * 2,
        ],
    )


def _edge_stage(t1, t2, c_all, dst, src, zeros, l):
    out = _make_edge_kernel(l)(t1, t2, c_all, dst, src, zeros)
    return out.reshape(NT, N, CW)


def kernel(x, edge_index, edge_attr, batch, W_emb, b_emb, Wf, bf, Ws, bs,
           gamma, beta, W1, b1, W2, b2, W3, b3):
    f32 = jnp.float32
    src = edge_index[0].astype(jnp.int32)
    dst = edge_index[1].astype(jnp.int32)

    wt1, wt2, wc, bc = [], [], [], []
    for l in range(L):
        w_f, w_s = Wf[l], Ws[l]
        q = lambda w, r0, t: w[r0:r0 + H, t * CW:(t + 1) * CW]
        wt1.append(jnp.stack([
            jnp.concatenate([q(w_f, 0, t), q(w_s, 0, t)], axis=1)
            for t in range(NT)]))
        wt2.append(jnp.stack([
            jnp.concatenate([q(w_f, H, t), q(w_s, H, t)], axis=1)
            for t in range(NT)]))
        wc.append(jnp.stack([
            jnp.concatenate([w_f[2 * H:, t * CW:(t + 1) * CW],
                             w_s[2 * H:, t * CW:(t + 1) * CW]], axis=1)
            for t in range(NT)]))
        bc.append(jnp.stack([
            jnp.concatenate([bf[l][t * CW:(t + 1) * CW],
                             bs[l][t * CW:(t + 1) * CW]])
            for t in range(NT)])[:, None, :])
    wt1 = jnp.stack(wt1)
    wt2 = jnp.stack(wt2)
    wc = jnp.stack(wc)
    bc = jnp.stack(bc)
    wcat = jnp.concatenate(
        [wc[l, p * 2 + c2] for c2 in range(2) for l in range(L)
         for p in range(2)], axis=1)
    bcat = jnp.concatenate(
        [bc[l, p * 2 + c2] for c2 in range(2) for l in range(L)
         for p in range(2)], axis=1)

    c_all = _cprep(edge_attr.astype(f32), wcat, bcat)
    zeros = jnp.zeros((ZCH, CW), f32)
    batch3 = batch.astype(jnp.int32).reshape(NB_N, 1, BN)

    h, t1, t2 = _embed(x.astype(f32), W_emb, b_emb.reshape(1, H),
                       wt1[0], wt2[0])
    for l in range(L):
        agg = _edge_stage(t1.reshape(NT * N, TW), t2.reshape(NT * N, TW),
                          c_all, dst, src, zeros, l)
        st = _stats(agg)
        gam = gamma[l].reshape(1, H)
        bet = beta[l].reshape(1, H)
        if l < L - 1:
            h, t1, t2 = _apply_tables(agg, h, st, gam, bet,
                                      wt1[l + 1], wt2[l + 1])
        else:
            h = _apply_last(agg, h, st, gam, bet)

    sums, cnt = _pool(h, batch3)
    return _head(sums, cnt, W1, b1.reshape(1, 128), W2, b2.reshape(1, 64),
                 W3, b3.reshape(1, 1))


# speedup vs baseline: 4.5783x; 1.0702x over previous
import functools

import jax
import jax.numpy as jnp
from jax import lax
from jax.experimental import pallas as pl
from jax.experimental.pallas import tpu as pltpu
from jax.experimental.pallas import tpu_sc as plsc

N = 50000
E = 800000
FEA = 92
H = 64
EDIM = 41
L = 4
G = 256
CW = 16
TW = 2 * CW
NT = 4

K = 256
NCH = E // K
NSUB = 16
ROWS_PER_TILE = N // NSUB
ZCH = 625
MAXJ = (NCH + NSUB - 1) // NSUB

BE = 1000
BN = 400
NB_N = N // BN

_LOG1P_C = (
    0.9999751958009936, -0.4993836513699655, 0.32778470930088355,
    -0.22478407188994595, 0.13329712681080494, -0.0543128720320267,
    0.010571220016300703,
)


def _poly_log1p(u):
    r = jnp.float32(_LOG1P_C[-1])
    for c in _LOG1P_C[-2::-1]:
        r = r * u + jnp.float32(c)
    return r * u


_EXP_C = (1.0 / 720, 1.0 / 120, 1.0 / 24, 1.0 / 6, 0.5, 1.0, 1.0)


def _sc_exp(x):
    t = x * jnp.float32(1.4426950408889634)
    t = jnp.minimum(jnp.maximum(t, jnp.float32(-126.0)), jnp.float32(126.0))
    half = jnp.where(t >= 0, jnp.float32(0.5), jnp.float32(-0.5))
    n = (t + half).astype(jnp.int32)
    y = (t - n.astype(jnp.float32)) * jnp.float32(0.6931471805599453)
    p = jnp.float32(_EXP_C[0])
    for c in _EXP_C[1:]:
        p = p * y + jnp.float32(c)
    scale = lax.bitcast_convert_type((n + 127) << 23, jnp.float32)
    return p * scale


def _sc_rcp(d):
    r = lax.bitcast_convert_type(
        jnp.int32(0x7EF311C3) - lax.bitcast_convert_type(d, jnp.int32),
        jnp.float32)
    for _ in range(3):
        r = r * (jnp.float32(2.0) - d * r)
    return r


def _sc_softplus(x):
    u = jnp.exp(-jnp.abs(x))
    return jnp.maximum(x, jnp.float32(0.0)) + _poly_log1p(u)


def _sc_sigmoid(x):
    return jnp.float32(1.0) / (jnp.float32(1.0) + jnp.exp(-x))


def _tc_softplus(x):
    return jnp.maximum(x, 0.0) + jnp.log1p(jnp.exp(-jnp.abs(x)))


_CJ = 2 * L * TW


def _cprep_body(ea_ref, w_ref, b_ref, out_ref):
    r = jnp.dot(ea_ref[...], w_ref[...], preferred_element_type=jnp.float32,
                precision=lax.Precision.HIGHEST) + b_ref[...]
    out_ref[0] = r[:, :_CJ]
    out_ref[1] = r[:, _CJ:]


def _cprep(ea, wcat, bcat):
    return pl.pallas_call(
        _cprep_body,
        grid=(E // BE,),
        in_specs=[
            pl.BlockSpec((BE, EDIM), lambda i: (i, 0)),
            pl.BlockSpec((EDIM, 2 * _CJ), lambda i: (0, 0)),
            pl.BlockSpec((1, 2 * _CJ), lambda i: (0, 0)),
        ],
        out_specs=pl.BlockSpec((2, BE, _CJ), lambda i: (0, i, 0)),
        out_shape=jax.ShapeDtypeStruct((2, E, _CJ), jnp.float32),
    )(ea, wcat, bcat)


def _embed_body(x_ref, we_ref, be_ref, wt1_ref, wt2_ref, h_ref, t1_ref, t2_ref):
    h = jnp.dot(x_ref[...], we_ref[...], preferred_element_type=jnp.float32,
                        precision=lax.Precision.HIGHEST)
    h = h + be_ref[...]
    h_ref[...] = h
    for t in range(NT):
        t1_ref[t] = jnp.dot(h, wt1_ref[t], preferred_element_type=jnp.float32,
                        precision=lax.Precision.HIGHEST)
        t2_ref[t] = jnp.dot(h, wt2_ref[t], preferred_element_type=jnp.float32,
                        precision=lax.Precision.HIGHEST)


def _embed(x, we, be, wt1, wt2):
    return pl.pallas_call(
        _embed_body,
        grid=(NB_N,),
        in_specs=[
            pl.BlockSpec((BN, FEA), lambda i: (i, 0)),
            pl.BlockSpec((FEA, H), lambda i: (0, 0)),
            pl.BlockSpec((1, H), lambda i: (0, 0)),
            pl.BlockSpec((NT, H, TW), lambda i: (0, 0, 0)),
            pl.BlockSpec((NT, H, TW), lambda i: (0, 0, 0)),
        ],
        out_specs=[
            pl.BlockSpec((BN, H), lambda i: (i, 0)),
            pl.BlockSpec((NT, BN, TW), lambda i: (0, i, 0)),
            pl.BlockSpec((NT, BN, TW), lambda i: (0, i, 0)),
        ],
        out_shape=[
            jax.ShapeDtypeStruct((N, H), jnp.float32),
            jax.ShapeDtypeStruct((NT, N, TW), jnp.float32),
            jax.ShapeDtypeStruct((NT, N, TW), jnp.float32),
        ],
    )(x, we, be, wt1, wt2)


def _stats_body(agg_ref, out_ref):
    i = pl.program_id(0)
    a = agg_ref[...]
    ssum = jnp.sum(a, axis=1)
    ssq = jnp.sum(a * a, axis=1)

    @pl.when(i == 0)
    def _():
        out_ref[0] = ssum
        out_ref[1] = ssq

    @pl.when(i > 0)
    def _():
        out_ref[0] += ssum
        out_ref[1] += ssq


def _stats(agg):
    return pl.pallas_call(
        _stats_body,
        grid=(NB_N,),
        in_specs=[pl.BlockSpec((NT, BN, CW), lambda i: (0, i, 0))],
        out_specs=pl.BlockSpec((2, NT, CW), lambda i: (0, 0, 0)),
        out_shape=jax.ShapeDtypeStruct((2, NT, CW), jnp.float32),
    )(agg)


def _bn_from_stats(stats_ref, agg_blk):
    ssum = stats_ref[0].reshape(1, H)
    ssq = stats_ref[1].reshape(1, H)
    mu = ssum * jnp.float32(1.0 / N)
    var = ssq * jnp.float32(1.0 / N) - mu * mu
    inv = lax.rsqrt(var + jnp.float32(1e-5))
    aggf = jnp.concatenate([agg_blk[t] for t in range(NT)], axis=1)
    return (aggf - mu) * inv


def _applyt_body(agg_ref, h_ref, st_ref, gam_ref, bet_ref, wt1_ref, wt2_ref,
                 hn_ref, t1_ref, t2_ref):
    norm = _bn_from_stats(st_ref, agg_ref[...])
    hn = h_ref[...] + gam_ref[...] * norm + bet_ref[...]
    hn_ref[...] = hn
    for t in range(NT):
        t1_ref[t] = jnp.dot(hn, wt1_ref[t], preferred_element_type=jnp.float32,
                        precision=lax.Precision.HIGHEST)
        t2_ref[t] = jnp.dot(hn, wt2_ref[t], preferred_element_type=jnp.float32,
                        precision=lax.Precision.HIGHEST)


def _apply_tables(agg, h, st, gam, bet, wt1, wt2):
    return pl.pallas_call(
        _applyt_body,
        grid=(NB_N,),
        in_specs=[
            pl.BlockSpec((NT, BN, CW), lambda i: (0, i, 0)),
            pl.BlockSpec((BN, H), lambda i: (i, 0)),
            pl.BlockSpec((2, NT, CW), lambda i: (0, 0, 0)),
            pl.BlockSpec((1, H), lambda i: (0, 0)),
            pl.BlockSpec((1, H), lambda i: (0, 0)),
            pl.BlockSpec((NT, H, TW), lambda i: (0, 0, 0)),
            pl.BlockSpec((NT, H, TW), lambda i: (0, 0, 0)),
        ],
        out_specs=[
            pl.BlockSpec((BN, H), lambda i: (i, 0)),
            pl.BlockSpec((NT, BN, TW), lambda i: (0, i, 0)),
            pl.BlockSpec((NT, BN, TW), lambda i: (0, i, 0)),
        ],
        out_shape=[
            jax.ShapeDtypeStruct((N, H), jnp.float32),
            jax.ShapeDtypeStruct((NT, N, TW), jnp.float32),
            jax.ShapeDtypeStruct((NT, N, TW), jnp.float32),
        ],
    )(agg, h, st, gam, bet, wt1, wt2)


def _applyl_body(agg_ref, h_ref, st_ref, gam_ref, bet_ref, hn_ref):
    norm = _bn_from_stats(st_ref, agg_ref[...])
    hn_ref[...] = h_ref[...] + gam_ref[...] * norm + bet_ref[...]


def _apply_last(agg, h, st, gam, bet):
    return pl.pallas_call(
        _applyl_body,
        grid=(NB_N,),
        in_specs=[
            pl.BlockSpec((NT, BN, CW), lambda i: (0, i, 0)),
            pl.BlockSpec((BN, H), lambda i: (i, 0)),
            pl.BlockSpec((2, NT, CW), lambda i: (0, 0, 0)),
            pl.BlockSpec((1, H), lambda i: (0, 0)),
            pl.BlockSpec((1, H), lambda i: (0, 0)),
        ],
        out_specs=pl.BlockSpec((BN, H), lambda i: (i, 0)),
        out_shape=jax.ShapeDtypeStruct((N, H), jnp.float32),
    )(agg, h, st, gam, bet)


def _pool_body(h_ref, b_ref, sums_ref, cnt_ref):
    i = pl.program_id(0)
    hblk = h_ref[...]
    ids = b_ref[0]
    gids = lax.broadcasted_iota(jnp.int32, (G, BN), 0)
    oh = (gids == ids).astype(jnp.float32)
    s = jnp.dot(oh, hblk, preferred_element_type=jnp.float32,
                        precision=lax.Precision.HIGHEST)
    c = jnp.sum(oh, axis=1, keepdims=True)

    @pl.when(i == 0)
    def _():
        sums_ref[...] = s
        cnt_ref[...] = c

    @pl.when(i > 0)
    def _():
        sums_ref[...] += s
        cnt_ref[...] += c


def _pool(h, batch3):
    return pl.pallas_call(
        _pool_body,
        grid=(NB_N,),
        in_specs=[
            pl.BlockSpec((BN, H), lambda i: (i, 0)),
            pl.BlockSpec((1, 1, BN), lambda i: (i, 0, 0)),
        ],
        out_specs=[
            pl.BlockSpec((G, H), lambda i: (0, 0)),
            pl.BlockSpec((G, 1), lambda i: (0, 0)),
        ],
        out_shape=[
            jax.ShapeDtypeStruct((G, H), jnp.float32),
            jax.ShapeDtypeStruct((G, 1), jnp.float32),
        ],
    )(h, batch3)


def _head_body(s_ref, c_ref, w1_ref, b1_ref, w2_ref, b2_ref, w3_ref, b3_ref,
               out_ref):
    pooled = s_ref[...] / jnp.maximum(c_ref[...], 1.0)
    y = _tc_softplus(
        jnp.dot(pooled, w1_ref[...], preferred_element_type=jnp.float32,
                        precision=lax.Precision.HIGHEST)
        + b1_ref[...])
    y = _tc_softplus(
        jnp.dot(y, w2_ref[...], preferred_element_type=jnp.float32,
                        precision=lax.Precision.HIGHEST)
        + b2_ref[...])
    out_ref[...] = (
        jnp.dot(y, w3_ref[...], preferred_element_type=jnp.float32,
                        precision=lax.Precision.HIGHEST)
        + b3_ref[...])


def _head(sums, cnt, w1, b1, w2, b2, w3, b3):
    return pl.pallas_call(
        _head_body,
        out_shape=jax.ShapeDtypeStruct((G, 1), jnp.float32),
    )(sums, cnt, w1, b1, w2, b2, w3, b3)


def _edge_body(l, t1_hbm, t2_hbm, c_hbm, dst_hbm, src_hbm, z_hbm, out_hbm,
               idxd, idxs, idxa1, idxa2, g1, g2, cb, mb, accum, semg):
    c = lax.axis_index("c")
    s = lax.axis_index("s")
    rows0 = s * ROWS_PER_TILE

    for p in range(2):
        for k in range(ROWS_PER_TILE // ZCH):
            pltpu.sync_copy(z_hbm, accum.at[pl.ds(rows0 + k * ZCH, ZCH)])
        plsc.subcore_barrier()

        tN = (p * 2) * N + c * N
        col0 = (l * 2 + p) * TW

        def issue(j, b):
            ch = s + NSUB * j

            @pl.when(ch < NCH)
            def _():
                off = ch * K
                pltpu.sync_copy(dst_hbm.at[pl.ds(off, K)], idxd[b])
                pltpu.sync_copy(src_hbm.at[pl.ds(off, K)], idxs)

                @plsc.parallel_loop(0, K, 16, unroll=4)
                def _adj(i):
                    idxa1[b][pl.ds(i, 16)] = idxd[b][pl.ds(i, 16)] + tN
                    idxa2[b][pl.ds(i, 16)] = idxs[pl.ds(i, 16)] + tN

                pltpu.async_copy(t1_hbm.at[idxa1[b]], g1[b], semg[b])
                pltpu.async_copy(t2_hbm.at[idxa2[b]], g2[b], semg[b])
                pltpu.async_copy(
                    c_hbm.at[c, pl.ds(off, K), pl.ds(col0, TW)],
                    cb[b], semg[b])

        issue(jnp.int32(0), 0)
        issue(jnp.int32(1), 1)

        def outer(jj, carry):
            for b in range(2):
                j = jj * 2 + b
                ch = s + NSUB * j

                @pl.when(ch < NCH)
                def _():
                    pltpu.make_async_copy(
                        t1_hbm.at[idxa1[b]], g1[b], semg[b]).wait()
                    pltpu.make_async_copy(
                        t2_hbm.at[idxa2[b]], g2[b], semg[b]).wait()
                    pltpu.make_async_copy(
                        c_hbm.at[c, pl.ds(ch * K, K), pl.ds(col0, TW)],
                        cb[b], semg[b]).wait()

                    @plsc.parallel_loop(0, K, 1, unroll=8)
                    def _edge(e):
                        pf = (g1[b][e, pl.ds(0, CW)] + g2[b][e, pl.ds(0, CW)]
                              + cb[b][e, pl.ds(0, CW)])
                        ps = (g1[b][e, pl.ds(CW, CW)]
                              + g2[b][e, pl.ds(CW, CW)]
                              + cb[b][e, pl.ds(CW, CW)])
                        mb[e, pl.ds(0, CW)] = _sc_sigmoid(pf) * _sc_softplus(ps)

                    pltpu.sync_copy(mb, accum.at[idxd[b]], add=True)
                    issue(j + 2, b)

            return carry

        lax.fori_loop(0, MAXJ // 2, outer, 0)
        plsc.subcore_barrier()
        pltpu.sync_copy(accum.at[pl.ds(rows0, ROWS_PER_TILE)],
                        out_hbm.at[p, c, s])


@functools.lru_cache(maxsize=None)
def _make_edge_kernel(l):
    return pl.kernel(
        functools.partial(_edge_body, l),
        mesh=plsc.VectorSubcoreMesh(core_axis_name="c", subcore_axis_name="s"),
        compiler_params=pltpu.CompilerParams(use_tc_tiling_on_sc=False),
        out_type=jax.ShapeDtypeStruct((2, 2, NSUB, ROWS_PER_TILE, CW),
                                      jnp.float32),
        scratch_types=[
            [pltpu.VMEM((K,), jnp.int32)] * 2,
            pltpu.VMEM((K,), jnp.int32),
            [pltpu.VMEM((K,), jnp.int32)] * 2,
            [pltpu.VMEM((K,), jnp.int32)] * 2,
            [pltpu.VMEM((K, TW), jnp.float32)] * 2,
            [pltpu.VMEM((K, TW), jnp.float32)] * 2,
            [pltpu.VMEM((K, TW), jnp.float32)] * 2,
            pltpu.VMEM((K, CW), jnp.float32),
            pltpu.VMEM_SHARED((N, CW), jnp.float32),
            [pltpu.SemaphoreType.DMA] * 2,
        ],
    )


def _edge_stage(t1, t2, c_all, dst, src, zeros, l):
    out = _make_edge_kernel(l)(t1, t2, c_all, dst, src, zeros)
    return out.reshape(NT, N, CW)


def kernel(x, edge_index, edge_attr, batch, W_emb, b_emb, Wf, bf, Ws, bs,
           gamma, beta, W1, b1, W2, b2, W3, b3):
    f32 = jnp.float32
    src = edge_index[0].astype(jnp.int32)
    dst = edge_index[1].astype(jnp.int32)

    wt1, wt2, wc, bc = [], [], [], []
    for l in range(L):
        w_f, w_s = Wf[l], Ws[l]
        q = lambda w, r0, t: w[r0:r0 + H, t * CW:(t + 1) * CW]
        wt1.append(jnp.stack([
            jnp.concatenate([q(w_f, 0, t), q(w_s, 0, t)], axis=1)
            for t in range(NT)]))
        wt2.append(jnp.stack([
            jnp.concatenate([q(w_f, H, t), q(w_s, H, t)], axis=1)
            for t in range(NT)]))
        wc.append(jnp.stack([
            jnp.concatenate([w_f[2 * H:, t * CW:(t + 1) * CW],
                             w_s[2 * H:, t * CW:(t + 1) * CW]], axis=1)
            for t in range(NT)]))
        bc.append(jnp.stack([
            jnp.concatenate([bf[l][t * CW:(t + 1) * CW],
                             bs[l][t * CW:(t + 1) * CW]])
            for t in range(NT)])[:, None, :])
    wt1 = jnp.stack(wt1)
    wt2 = jnp.stack(wt2)
    wc = jnp.stack(wc)
    bc = jnp.stack(bc)
    wcat = jnp.concatenate(
        [wc[l, p * 2 + c2] for c2 in range(2) for l in range(L)
         for p in range(2)], axis=1)
    bcat = jnp.concatenate(
        [bc[l, p * 2 + c2] for c2 in range(2) for l in range(L)
         for p in range(2)], axis=1)

    c_all = _cprep(edge_attr.astype(f32), wcat, bcat)
    zeros = jnp.zeros((ZCH, CW), f32)
    batch3 = batch.astype(jnp.int32).reshape(NB_N, 1, BN)

    h, t1, t2 = _embed(x.astype(f32), W_emb, b_emb.reshape(1, H),
                       wt1[0], wt2[0])
    for l in range(L):
        agg = _edge_stage(t1.reshape(NT * N, TW), t2.reshape(NT * N, TW),
                          c_all, dst, src, zeros, l)
        st = _stats(agg)
        gam = gamma[l].reshape(1, H)
        bet = beta[l].reshape(1, H)
        if l < L - 1:
            h, t1, t2 = _apply_tables(agg, h, st, gam, bet,
                                      wt1[l + 1], wt2[l + 1])
        else:
            h = _apply_last(agg, h, st, gam, bet)

    sums, cnt = _pool(h, batch3)
    return _head(sums, cnt, W1, b1.reshape(1, 128), W2, b2.reshape(1, 64),
                 W3, b3.reshape(1, 1))
